# baseline (speedup 1.0000x reference)
import functools

import jax
import jax.numpy as jnp
from jax import lax
from jax.experimental import pallas as pl
from jax.experimental.pallas import tpu as pltpu
from jax.experimental.pallas import tpu_sc as plsc

B, A, M = 64, 128, 12
ORIG, NBR, AF, NC = 92, 41, 64, 3
N = B * A
K = N * M
F2 = 2 * AF

_SC_CORES, _SC_SUBCORES = 2, 16
_NW = _SC_CORES * _SC_SUBCORES
_ROWS_PER_W = K // _NW
_CH = 128
_NCH = _ROWS_PER_W // _CH


def _sc_gather(table, idx):
    mesh = plsc.VectorSubcoreMesh(core_axis_name="c", subcore_axis_name="s")

    @functools.partial(
        pl.kernel, mesh=mesh,
        out_type=jax.ShapeDtypeStruct((K, F2), jnp.float32),
        scratch_types=[
            pltpu.VMEM((_CH,), jnp.int32),
            pltpu.VMEM((_CH, F2), jnp.float32),
            pltpu.SemaphoreType.DMA,
        ],
    )
    def gk(table_hbm, idx_hbm, out_hbm, idx_v, rows_v, sem):
        wid = lax.axis_index("s") * _SC_CORES + lax.axis_index("c")

        def body(j, carry):
            base = pl.multiple_of(wid * _ROWS_PER_W + j * _CH, _CH)
            pltpu.sync_copy(idx_hbm.at[pl.ds(base, _CH)], idx_v)
            pltpu.async_copy(table_hbm.at[idx_v], rows_v, sem).wait()
            pltpu.sync_copy(rows_v, out_hbm.at[pl.ds(base, _CH)])
            return carry

        lax.fori_loop(0, _NCH, body, 0)

    return gk(table, idx)


def _softplus(z):
    return jnp.log(1.0 + jnp.exp(-jnp.abs(z))) + jnp.maximum(z, 0.0)


def _sigmoid(z):
    return 1.0 / (1.0 + jnp.exp(-z))


_RT = 1024
_NT = 128
_GT = _NT * M


def _embed_body(af_ref, wemb_ref, wcat_ref, x_ref, ps_ref, pn_ref):
    x = jnp.dot(af_ref[...], wemb_ref[...], preferred_element_type=jnp.float32)
    x_ref[...] = x
    p = jnp.dot(x, wcat_ref[...], preferred_element_type=jnp.float32)
    ps_ref[...] = p[:, :F2]
    pn_ref[...] = p[:, F2:]


def _embed(atom_fea, W_emb, Wcat):
    return pl.pallas_call(
        _embed_body,
        grid=(N // _RT,),
        in_specs=[
            pl.BlockSpec((_RT, ORIG), lambda i: (i, 0)),
            pl.BlockSpec((ORIG, AF), lambda i: (0, 0)),
            pl.BlockSpec((AF, 2 * F2), lambda i: (0, 0)),
        ],
        out_specs=[
            pl.BlockSpec((_RT, AF), lambda i: (i, 0)),
            pl.BlockSpec((_RT, F2), lambda i: (i, 0)),
            pl.BlockSpec((_RT, F2), lambda i: (i, 0)),
        ],
        out_shape=[
            jax.ShapeDtypeStruct((N, AF), jnp.float32),
            jax.ShapeDtypeStruct((N, F2), jnp.float32),
            jax.ShapeDtypeStruct((N, F2), jnp.float32),
        ],
    )(atom_fea, W_emb, Wcat)


def _g_tile(an_ref, nbr_ref, ps_ref, wfe_ref):
    pe = jnp.dot(nbr_ref[...], wfe_ref[...], preferred_element_type=jnp.float32)
    ps = ps_ref[...]
    psr = jnp.broadcast_to(ps[:, None, :], (_NT, M, F2)).reshape(_GT, F2)
    return pe + an_ref[...] + psr


def _stats_body(an_ref, nbr_ref, ps_ref, wfe_ref, su_ref, sq_ref):
    g = _g_tile(an_ref, nbr_ref, ps_ref, wfe_ref)

    @pl.when(pl.program_id(0) == 0)
    def _():
        su_ref[...] = jnp.zeros_like(su_ref)
        sq_ref[...] = jnp.zeros_like(sq_ref)

    su_ref[...] += jnp.sum(g, axis=0, keepdims=True)
    sq_ref[...] += jnp.sum(g * g, axis=0, keepdims=True)


def _stats(an, nbr_flat, ps, Wfe):
    return pl.pallas_call(
        _stats_body,
        grid=(N // _NT,),
        in_specs=[
            pl.BlockSpec((_GT, F2), lambda i: (i, 0)),
            pl.BlockSpec((_GT, NBR), lambda i: (i, 0)),
            pl.BlockSpec((_NT, F2), lambda i: (i, 0)),
            pl.BlockSpec((NBR, F2), lambda i: (0, 0)),
        ],
        out_specs=[
            pl.BlockSpec((1, F2), lambda i: (0, 0)),
            pl.BlockSpec((1, F2), lambda i: (0, 0)),
        ],
        out_shape=[
            jax.ShapeDtypeStruct((1, F2), jnp.float32),
            jax.ShapeDtypeStruct((1, F2), jnp.float32),
        ],
    )(an, nbr_flat, ps, Wfe)


def _act_body(an_ref, nbr_ref, ps_ref, wfe_ref, su_ref, sq_ref, g1_ref, bt1_ref,
              s_ref, ssu_ref, ssq_ref):
    mean = su_ref[...] / K
    var = sq_ref[...] / K - mean * mean
    a = g1_ref[...] * lax.rsqrt(var + 1e-5)
    c = bt1_ref[...] - mean * a
    g = _g_tile(an_ref, nbr_ref, ps_ref, wfe_ref)
    gh = g * a + c
    fl = _sigmoid(gh[:, :AF])
    co = _softplus(gh[:, AF:])
    s = jnp.sum((fl * co).reshape(_NT, M, AF), axis=1)
    s_ref[...] = s

    @pl.when(pl.program_id(0) == 0)
    def _():
        ssu_ref[...] = jnp.zeros_like(ssu_ref)
        ssq_ref[...] = jnp.zeros_like(ssq_ref)

    ssu_ref[...] += jnp.sum(s, axis=0, keepdims=True)
    ssq_ref[...] += jnp.sum(s * s, axis=0, keepdims=True)


def _act(an, nbr_flat, ps, Wfe, su, sq, g1, bt1):
    return pl.pallas_call(
        _act_body,
        grid=(N // _NT,),
        in_specs=[
            pl.BlockSpec((_GT, F2), lambda i: (i, 0)),
            pl.BlockSpec((_GT, NBR), lambda i: (i, 0)),
            pl.BlockSpec((_NT, F2), lambda i: (i, 0)),
            pl.BlockSpec((NBR, F2), lambda i: (0, 0)),
            pl.BlockSpec((1, F2), lambda i: (0, 0)),
            pl.BlockSpec((1, F2), lambda i: (0, 0)),
            pl.BlockSpec((1, F2), lambda i: (0, 0)),
            pl.BlockSpec((1, F2), lambda i: (0, 0)),
        ],
        out_specs=[
            pl.BlockSpec((_NT, AF), lambda i: (i, 0)),
            pl.BlockSpec((1, AF), lambda i: (0, 0)),
            pl.BlockSpec((1, AF), lambda i: (0, 0)),
        ],
        out_shape=[
            jax.ShapeDtypeStruct((N, AF), jnp.float32),
            jax.ShapeDtypeStruct((1, AF), jnp.float32),
            jax.ShapeDtypeStruct((1, AF), jnp.float32),
        ],
    )(an, nbr_flat, ps, Wfe, su, sq, g1, bt1)


def _bn2_update(x_ref, s_ref, ssu_ref, ssq_ref, g2_ref, bt2_ref):
    m2 = ssu_ref[...] / N
    v2 = ssq_ref[...] / N - m2 * m2
    a2 = g2_ref[...] * lax.rsqrt(v2 + 1e-5)
    c2 = bt2_ref[...] - m2 * a2
    return _softplus(x_ref[...] + s_ref[...] * a2 + c2)


def _update_body(x_ref, s_ref, ssu_ref, ssq_ref, g2_ref, bt2_ref, wcat_ref,
                 xo_ref, ps_ref, pn_ref):
    xn = _bn2_update(x_ref, s_ref, ssu_ref, ssq_ref, g2_ref, bt2_ref)
    xo_ref[...] = xn
    p = jnp.dot(xn, wcat_ref[...], preferred_element_type=jnp.float32)
    ps_ref[...] = p[:, :F2]
    pn_ref[...] = p[:, F2:]


def _update(x, s, ssu, ssq, g2, bt2, Wcat):
    return pl.pallas_call(
        _update_body,
        grid=(N // _RT,),
        in_specs=[
            pl.BlockSpec((_RT, AF), lambda i: (i, 0)),
            pl.BlockSpec((_RT, AF), lambda i: (i, 0)),
            pl.BlockSpec((1, AF), lambda i: (0, 0)),
            pl.BlockSpec((1, AF), lambda i: (0, 0)),
            pl.BlockSpec((1, AF), lambda i: (0, 0)),
            pl.BlockSpec((1, AF), lambda i: (0, 0)),
            pl.BlockSpec((AF, 2 * F2), lambda i: (0, 0)),
        ],
        out_specs=[
            pl.BlockSpec((_RT, AF), lambda i: (i, 0)),
            pl.BlockSpec((_RT, F2), lambda i: (i, 0)),
            pl.BlockSpec((_RT, F2), lambda i: (i, 0)),
        ],
        out_shape=[
            jax.ShapeDtypeStruct((N, AF), jnp.float32),
            jax.ShapeDtypeStruct((N, F2), jnp.float32),
            jax.ShapeDtypeStruct((N, F2), jnp.float32),
        ],
    )(x, s, ssu, ssq, g2, bt2, Wcat)


def _update_last_body(x_ref, s_ref, ssu_ref, ssq_ref, g2_ref, bt2_ref, xo_ref):
    xo_ref[...] = _bn2_update(x_ref, s_ref, ssu_ref, ssq_ref, g2_ref, bt2_ref)


def _update_last(x, s, ssu, ssq, g2, bt2):
    return pl.pallas_call(
        _update_last_body,
        grid=(N // _RT,),
        in_specs=[
            pl.BlockSpec((_RT, AF), lambda i: (i, 0)),
            pl.BlockSpec((_RT, AF), lambda i: (i, 0)),
            pl.BlockSpec((1, AF), lambda i: (0, 0)),
            pl.BlockSpec((1, AF), lambda i: (0, 0)),
            pl.BlockSpec((1, AF), lambda i: (0, 0)),
            pl.BlockSpec((1, AF), lambda i: (0, 0)),
        ],
        out_specs=pl.BlockSpec((_RT, AF), lambda i: (i, 0)),
        out_shape=jax.ShapeDtypeStruct((N, AF), jnp.float32),
    )(x, s, ssu, ssq, g2, bt2)


def _final_body(x_ref, w2_ref, b2_ref, waf_ref, baf_ref,
                ep_ref, af_ref, z_ref, n_ref):
    x = x_ref[...]
    nrm = jnp.sqrt(jnp.sum(x * x, axis=1, keepdims=True))
    nd = x / jnp.maximum(nrm, 1e-12)
    n_ref[...] = nd[None]
    z_ref[...] = jnp.mean(nd, axis=0, keepdims=True)[None]
    af_ref[...] = jnp.dot(nd, waf_ref[...], preferred_element_type=jnp.float32) + baf_ref[...]
    es = []
    for j in range(6):
        tmp = jnp.dot(nd, w2_ref[j], preferred_element_type=jnp.float32)
        e = lax.dot_general(tmp, nd, (((1,), (1,)), ((), ())),
                            preferred_element_type=jnp.float32)
        es.append(e + b2_ref[0, j])
    mx = es[0]
    for j in range(1, 6):
        mx = jnp.maximum(mx, es[j])
    se = jnp.exp(es[0] - mx)
    for j in range(1, 6):
        se += jnp.exp(es[j] - mx)
    off = mx + jnp.log(se)
    outs = [es[j] - off for j in range(6)]
    ep_ref[...] = jnp.stack(outs, axis=-1).reshape(A, A * 6)


def _final(x, W2, b2, W_af, b_af):
    return pl.pallas_call(
        _final_body,
        grid=(B,),
        in_specs=[
            pl.BlockSpec((A, AF), lambda i: (i, 0)),
            pl.BlockSpec((6, AF, AF), lambda i: (0, 0, 0)),
            pl.BlockSpec((1, 6), lambda i: (0, 0)),
            pl.BlockSpec((AF, ORIG), lambda i: (0, 0)),
            pl.BlockSpec((1, ORIG), lambda i: (0, 0)),
        ],
        out_specs=[
            pl.BlockSpec((A, A * 6), lambda i: (i, 0)),
            pl.BlockSpec((A, ORIG), lambda i: (i, 0)),
            pl.BlockSpec((1, 1, AF), lambda i: (i, 0, 0)),
            pl.BlockSpec((1, A, AF), lambda i: (i, 0, 0)),
        ],
        out_shape=[
            jax.ShapeDtypeStruct((N, A * 6), jnp.float32),
            jax.ShapeDtypeStruct((N, ORIG), jnp.float32),
            jax.ShapeDtypeStruct((B, 1, AF), jnp.float32),
            jax.ShapeDtypeStruct((B, A, AF), jnp.float32),
        ],
    )(x, W2, b2, W_af, b_af)


def kernel(atom_fea, nbr_fea, nbr_fea_idx, crystal_atom_idx, cuda_flag, W_emb,
           Wf0, bf0, g1_0, bt1_0, g2_0, bt2_0,
           Wf1, bf1, g1_1, bt1_1, g2_1, bt2_1,
           Wf2, bf2, g1_2, bt1_2, g2_2, bt2_2,
           W_bil, b_bil, W_fc1, b_fc1, W_af, b_af):
    Wf = [Wf0, Wf1, Wf2]
    g1 = [g1_0[None], g1_1[None], g1_2[None]]
    bt1 = [bt1_0[None], bt1_1[None], bt1_2[None]]
    g2 = [g2_0[None], g2_1[None], g2_2[None]]
    bt2 = [bt2_0[None], bt2_1[None], bt2_2[None]]
    Wcat = [jnp.concatenate([w[:AF], w[AF:2 * AF]], axis=1) for w in Wf]
    Wfe = [w[2 * AF:] for w in Wf]
    idx = nbr_fea_idx.reshape(-1).astype(jnp.int32)
    nbr_flat = nbr_fea.reshape(K, NBR)

    x, ps, pn = _embed(atom_fea, W_emb, Wcat[0])
    for l in range(NC):
        an = _sc_gather(pn, idx)
        su, sq = _stats(an, nbr_flat, ps, Wfe[l])
        s, ssu, ssq = _act(an, nbr_flat, ps, Wfe[l], su, sq, g1[l], bt1[l])
        if l + 1 < NC:
            x, ps, pn = _update(x, s, ssu, ssq, g2[l], bt2[l], Wcat[l + 1])
        else:
            x = _update_last(x, s, ssu, ssq, g2[l], bt2[l])

    W2 = jnp.einsum('kde,kj->jde', W_bil, W_fc1)
    b2 = (b_bil @ W_fc1 + b_fc1)[None]
    ep8, af, z, normed = _final(x, W2, b2, W_af, b_af[None])
    return ep8.reshape(-1, 6), af, z.reshape(B, AF), normed, x


# speedup vs baseline: 3.0743x; 3.0743x over previous
import functools

import jax
import jax.numpy as jnp
from jax import lax
from jax.experimental import pallas as pl
from jax.experimental.pallas import tpu as pltpu
from jax.experimental.pallas import tpu_sc as plsc

B, A, M = 64, 128, 12
ORIG, NBR, AF, NC = 92, 41, 64, 3
N = B * A
K = N * M
F2 = 2 * AF

_SC_CORES, _SC_SUBCORES = 2, 16
_NW = _SC_CORES * _SC_SUBCORES
_ROWS_PER_W = K // _NW
_CH = 128
_NCH = _ROWS_PER_W // _CH


def _sc_gather(table, idx):
    mesh = plsc.VectorSubcoreMesh(core_axis_name="c", subcore_axis_name="s")

    @functools.partial(
        pl.kernel, mesh=mesh,
        out_type=jax.ShapeDtypeStruct((K, F2), jnp.float32),
        scratch_types=[
            pltpu.VMEM((_CH,), jnp.int32),
            pltpu.VMEM((_CH, F2), jnp.float32),
            pltpu.SemaphoreType.DMA,
        ],
    )
    def gk(table_hbm, idx_hbm, out_hbm, idx_v, rows_v, sem):
        wid = lax.axis_index("s") * _SC_CORES + lax.axis_index("c")

        def body(j, carry):
            base = pl.multiple_of(wid * _ROWS_PER_W + j * _CH, _CH)
            pltpu.sync_copy(idx_hbm.at[pl.ds(base, _CH)], idx_v)
            pltpu.async_copy(table_hbm.at[idx_v], rows_v, sem).wait()
            pltpu.sync_copy(rows_v, out_hbm.at[pl.ds(base, _CH)])
            return carry

        lax.fori_loop(0, _NCH, body, 0)

    return gk(table, idx)


def _softplus(z):
    return jnp.log(1.0 + jnp.exp(-jnp.abs(z))) + jnp.maximum(z, 0.0)


def _sigmoid(z):
    return 1.0 / (1.0 + jnp.exp(-z))


_RT = 1024
_NT = 128
_GT = _NT * M


def _embed_body(af_ref, wemb_ref, wcat_ref, x_ref, ps_ref, pn_ref):
    x = jnp.dot(af_ref[...], wemb_ref[...], preferred_element_type=jnp.float32)
    x_ref[...] = x
    p = jnp.dot(x, wcat_ref[...], preferred_element_type=jnp.float32)
    ps_ref[...] = p[:, :F2]
    pn_ref[...] = p[:, F2:]


def _embed(atom_fea, W_emb, Wcat):
    return pl.pallas_call(
        _embed_body,
        grid=(N // _RT,),
        in_specs=[
            pl.BlockSpec((_RT, ORIG), lambda i: (i, 0)),
            pl.BlockSpec((ORIG, AF), lambda i: (0, 0)),
            pl.BlockSpec((AF, 2 * F2), lambda i: (0, 0)),
        ],
        out_specs=[
            pl.BlockSpec((_RT, AF), lambda i: (i, 0)),
            pl.BlockSpec((_RT, F2), lambda i: (i, 0)),
            pl.BlockSpec((_RT, F2), lambda i: (i, 0)),
        ],
        out_shape=[
            jax.ShapeDtypeStruct((N, AF), jnp.float32),
            jax.ShapeDtypeStruct((N, F2), jnp.float32),
            jax.ShapeDtypeStruct((N, F2), jnp.float32),
        ],
    )(atom_fea, W_emb, Wcat)


def _g_tile(an_ref, nbr_ref, ps_ref, wfe_ref):
    pe = jnp.dot(nbr_ref[...], wfe_ref[...], preferred_element_type=jnp.float32)
    ps = ps_ref[...]
    psr = jnp.broadcast_to(ps[:, None, :], (_NT, M, F2)).reshape(_GT, F2)
    return pe + an_ref[...] + psr


def _stats_body(an_ref, nbr_ref, ps_ref, wfe_ref, su_ref, sq_ref):
    g = _g_tile(an_ref, nbr_ref, ps_ref, wfe_ref)

    @pl.when(pl.program_id(0) == 0)
    def _():
        su_ref[...] = jnp.zeros_like(su_ref)
        sq_ref[...] = jnp.zeros_like(sq_ref)

    su_ref[...] += jnp.sum(g, axis=0, keepdims=True)
    sq_ref[...] += jnp.sum(g * g, axis=0, keepdims=True)


def _stats(an, nbr_flat, ps, Wfe):
    return pl.pallas_call(
        _stats_body,
        grid=(N // _NT,),
        in_specs=[
            pl.BlockSpec((_GT, F2), lambda i: (i, 0)),
            pl.BlockSpec((_GT, NBR), lambda i: (i, 0)),
            pl.BlockSpec((_NT, F2), lambda i: (i, 0)),
            pl.BlockSpec((NBR, F2), lambda i: (0, 0)),
        ],
        out_specs=[
            pl.BlockSpec((1, F2), lambda i: (0, 0)),
            pl.BlockSpec((1, F2), lambda i: (0, 0)),
        ],
        out_shape=[
            jax.ShapeDtypeStruct((1, F2), jnp.float32),
            jax.ShapeDtypeStruct((1, F2), jnp.float32),
        ],
    )(an, nbr_flat, ps, Wfe)


def _act_body(an_ref, nbr_ref, ps_ref, wfe_ref, su_ref, sq_ref, g1_ref, bt1_ref,
              s_ref, ssu_ref, ssq_ref):
    mean = su_ref[...] / K
    var = sq_ref[...] / K - mean * mean
    a = g1_ref[...] * lax.rsqrt(var + 1e-5)
    c = bt1_ref[...] - mean * a
    g = _g_tile(an_ref, nbr_ref, ps_ref, wfe_ref)
    gh = g * a + c
    fl = _sigmoid(gh[:, :AF])
    co = _softplus(gh[:, AF:])
    s = jnp.sum((fl * co).reshape(_NT, M, AF), axis=1)
    s_ref[...] = s

    @pl.when(pl.program_id(0) == 0)
    def _():
        ssu_ref[...] = jnp.zeros_like(ssu_ref)
        ssq_ref[...] = jnp.zeros_like(ssq_ref)

    ssu_ref[...] += jnp.sum(s, axis=0, keepdims=True)
    ssq_ref[...] += jnp.sum(s * s, axis=0, keepdims=True)


def _act(an, nbr_flat, ps, Wfe, su, sq, g1, bt1):
    return pl.pallas_call(
        _act_body,
        grid=(N // _NT,),
        in_specs=[
            pl.BlockSpec((_GT, F2), lambda i: (i, 0)),
            pl.BlockSpec((_GT, NBR), lambda i: (i, 0)),
            pl.BlockSpec((_NT, F2), lambda i: (i, 0)),
            pl.BlockSpec((NBR, F2), lambda i: (0, 0)),
            pl.BlockSpec((1, F2), lambda i: (0, 0)),
            pl.BlockSpec((1, F2), lambda i: (0, 0)),
            pl.BlockSpec((1, F2), lambda i: (0, 0)),
            pl.BlockSpec((1, F2), lambda i: (0, 0)),
        ],
        out_specs=[
            pl.BlockSpec((_NT, AF), lambda i: (i, 0)),
            pl.BlockSpec((1, AF), lambda i: (0, 0)),
            pl.BlockSpec((1, AF), lambda i: (0, 0)),
        ],
        out_shape=[
            jax.ShapeDtypeStruct((N, AF), jnp.float32),
            jax.ShapeDtypeStruct((1, AF), jnp.float32),
            jax.ShapeDtypeStruct((1, AF), jnp.float32),
        ],
    )(an, nbr_flat, ps, Wfe, su, sq, g1, bt1)


def _bn2_update(x_ref, s_ref, ssu_ref, ssq_ref, g2_ref, bt2_ref):
    m2 = ssu_ref[...] / N
    v2 = ssq_ref[...] / N - m2 * m2
    a2 = g2_ref[...] * lax.rsqrt(v2 + 1e-5)
    c2 = bt2_ref[...] - m2 * a2
    return _softplus(x_ref[...] + s_ref[...] * a2 + c2)


def _update_body(x_ref, s_ref, ssu_ref, ssq_ref, g2_ref, bt2_ref, wcat_ref,
                 xo_ref, ps_ref, pn_ref):
    xn = _bn2_update(x_ref, s_ref, ssu_ref, ssq_ref, g2_ref, bt2_ref)
    xo_ref[...] = xn
    p = jnp.dot(xn, wcat_ref[...], preferred_element_type=jnp.float32)
    ps_ref[...] = p[:, :F2]
    pn_ref[...] = p[:, F2:]


def _update(x, s, ssu, ssq, g2, bt2, Wcat):
    return pl.pallas_call(
        _update_body,
        grid=(N // _RT,),
        in_specs=[
            pl.BlockSpec((_RT, AF), lambda i: (i, 0)),
            pl.BlockSpec((_RT, AF), lambda i: (i, 0)),
            pl.BlockSpec((1, AF), lambda i: (0, 0)),
            pl.BlockSpec((1, AF), lambda i: (0, 0)),
            pl.BlockSpec((1, AF), lambda i: (0, 0)),
            pl.BlockSpec((1, AF), lambda i: (0, 0)),
            pl.BlockSpec((AF, 2 * F2), lambda i: (0, 0)),
        ],
        out_specs=[
            pl.BlockSpec((_RT, AF), lambda i: (i, 0)),
            pl.BlockSpec((_RT, F2), lambda i: (i, 0)),
            pl.BlockSpec((_RT, F2), lambda i: (i, 0)),
        ],
        out_shape=[
            jax.ShapeDtypeStruct((N, AF), jnp.float32),
            jax.ShapeDtypeStruct((N, F2), jnp.float32),
            jax.ShapeDtypeStruct((N, F2), jnp.float32),
        ],
    )(x, s, ssu, ssq, g2, bt2, Wcat)


def _update_last_body(x_ref, s_ref, ssu_ref, ssq_ref, g2_ref, bt2_ref, xo_ref):
    xo_ref[...] = _bn2_update(x_ref, s_ref, ssu_ref, ssq_ref, g2_ref, bt2_ref)


def _update_last(x, s, ssu, ssq, g2, bt2):
    return pl.pallas_call(
        _update_last_body,
        grid=(N // _RT,),
        in_specs=[
            pl.BlockSpec((_RT, AF), lambda i: (i, 0)),
            pl.BlockSpec((_RT, AF), lambda i: (i, 0)),
            pl.BlockSpec((1, AF), lambda i: (0, 0)),
            pl.BlockSpec((1, AF), lambda i: (0, 0)),
            pl.BlockSpec((1, AF), lambda i: (0, 0)),
            pl.BlockSpec((1, AF), lambda i: (0, 0)),
        ],
        out_specs=pl.BlockSpec((_RT, AF), lambda i: (i, 0)),
        out_shape=jax.ShapeDtypeStruct((N, AF), jnp.float32),
    )(x, s, ssu, ssq, g2, bt2)


def _final_body(x_ref, w2c_ref, b2_ref, waf_ref, baf_ref,
                ep_ref, af_ref, z_ref, n_ref):
    x = x_ref[...]
    nrm = jnp.sqrt(jnp.sum(x * x, axis=1, keepdims=True))
    nd = x / jnp.maximum(nrm, 1e-12)
    n_ref[...] = nd[None]
    z_ref[...] = jnp.mean(nd, axis=0, keepdims=True)[None]
    af_ref[...] = jnp.dot(nd, waf_ref[...], preferred_element_type=jnp.float32) + baf_ref[...]
    tmp = jnp.dot(nd, w2c_ref[...], preferred_element_type=jnp.float32)
    tmp_r = jnp.concatenate([tmp[:, j * AF:(j + 1) * AF] for j in range(6)], axis=0)
    es_all = lax.dot_general(tmp_r, nd, (((1,), (1,)), ((), ())),
                             preferred_element_type=jnp.float32)
    es = [es_all[j * A:(j + 1) * A, :] + b2_ref[0, j] for j in range(6)]
    mx = es[0]
    for j in range(1, 6):
        mx = jnp.maximum(mx, es[j])
    se = jnp.exp(es[0] - mx)
    for j in range(1, 6):
        se += jnp.exp(es[j] - mx)
    off = mx + jnp.log(se)
    ep_ref[...] = jnp.concatenate([es[j] - off for j in range(6)], axis=0)[None]


def _final(x, W2c, b2, W_af, b_af):
    return pl.pallas_call(
        _final_body,
        grid=(B,),
        in_specs=[
            pl.BlockSpec((A, AF), lambda i: (i, 0)),
            pl.BlockSpec((AF, 6 * AF), lambda i: (0, 0)),
            pl.BlockSpec((1, 6), lambda i: (0, 0)),
            pl.BlockSpec((AF, ORIG), lambda i: (0, 0)),
            pl.BlockSpec((1, ORIG), lambda i: (0, 0)),
        ],
        out_specs=[
            pl.BlockSpec((1, 6 * A, A), lambda i: (i, 0, 0)),
            pl.BlockSpec((A, ORIG), lambda i: (i, 0)),
            pl.BlockSpec((1, 1, AF), lambda i: (i, 0, 0)),
            pl.BlockSpec((1, A, AF), lambda i: (i, 0, 0)),
        ],
        out_shape=[
            jax.ShapeDtypeStruct((B, 6 * A, A), jnp.float32),
            jax.ShapeDtypeStruct((N, ORIG), jnp.float32),
            jax.ShapeDtypeStruct((B, 1, AF), jnp.float32),
            jax.ShapeDtypeStruct((B, A, AF), jnp.float32),
        ],
    )(x, W2c, b2, W_af, b_af)


def kernel(atom_fea, nbr_fea, nbr_fea_idx, crystal_atom_idx, cuda_flag, W_emb,
           Wf0, bf0, g1_0, bt1_0, g2_0, bt2_0,
           Wf1, bf1, g1_1, bt1_1, g2_1, bt2_1,
           Wf2, bf2, g1_2, bt1_2, g2_2, bt2_2,
           W_bil, b_bil, W_fc1, b_fc1, W_af, b_af):
    Wf = [Wf0, Wf1, Wf2]
    g1 = [g1_0[None], g1_1[None], g1_2[None]]
    bt1 = [bt1_0[None], bt1_1[None], bt1_2[None]]
    g2 = [g2_0[None], g2_1[None], g2_2[None]]
    bt2 = [bt2_0[None], bt2_1[None], bt2_2[None]]
    Wcat = [jnp.concatenate([w[:AF], w[AF:2 * AF]], axis=1) for w in Wf]
    Wfe = [w[2 * AF:] for w in Wf]
    idx = nbr_fea_idx.reshape(-1).astype(jnp.int32)
    nbr_flat = nbr_fea.reshape(K, NBR)

    x, ps, pn = _embed(atom_fea, W_emb, Wcat[0])
    for l in range(NC):
        an = _sc_gather(pn, idx)
        su, sq = _stats(an, nbr_flat, ps, Wfe[l])
        s, ssu, ssq = _act(an, nbr_flat, ps, Wfe[l], su, sq, g1[l], bt1[l])
        if l + 1 < NC:
            x, ps, pn = _update(x, s, ssu, ssq, g2[l], bt2[l], Wcat[l + 1])
        else:
            x = _update_last(x, s, ssu, ssq, g2[l], bt2[l])

    W2 = jnp.einsum('kde,kj->jde', W_bil, W_fc1)
    W2c = jnp.concatenate([W2[j] for j in range(6)], axis=1)
    b2 = (b_bil @ W_fc1 + b_fc1)[None]
    epk, af, z, normed = _final(x, W2c, b2, W_af, b_af[None])
    ep = jnp.transpose(epk.reshape(B, 6, A, A), (0, 2, 3, 1)).reshape(-1, 6)
    return ep, af, z.reshape(B, AF), normed, x


# speedup vs baseline: 3.2960x; 1.0721x over previous
import functools

import jax
import jax.numpy as jnp
from jax import lax
from jax.experimental import pallas as pl
from jax.experimental.pallas import tpu as pltpu
from jax.experimental.pallas import tpu_sc as plsc

B, A, M = 64, 128, 12
ORIG, NBR, AF, NC = 92, 41, 64, 3
N = B * A
K = N * M
F2 = 2 * AF

_SC_CORES, _SC_SUBCORES = 2, 16
_NW = _SC_CORES * _SC_SUBCORES
_ROWS_PER_W = K // _NW
_CH = 128
_NCH = _ROWS_PER_W // _CH


def _sc_gather(table, idx):
    mesh = plsc.VectorSubcoreMesh(core_axis_name="c", subcore_axis_name="s")

    @functools.partial(
        pl.kernel, mesh=mesh,
        out_type=jax.ShapeDtypeStruct((K, F2), jnp.float32),
        scratch_types=[
            pltpu.VMEM((_CH,), jnp.int32),
            pltpu.VMEM((_CH, F2), jnp.float32),
            pltpu.SemaphoreType.DMA,
        ],
    )
    def gk(table_hbm, idx_hbm, out_hbm, idx_v, rows_v, sem):
        wid = lax.axis_index("s") * _SC_CORES + lax.axis_index("c")

        def body(j, carry):
            base = pl.multiple_of(wid * _ROWS_PER_W + j * _CH, _CH)
            pltpu.sync_copy(idx_hbm.at[pl.ds(base, _CH)], idx_v)
            pltpu.async_copy(table_hbm.at[idx_v], rows_v, sem).wait()
            pltpu.sync_copy(rows_v, out_hbm.at[pl.ds(base, _CH)])
            return carry

        lax.fori_loop(0, _NCH, body, 0)

    return gk(table, idx)


def _softplus(z):
    return jnp.log(1.0 + jnp.exp(-jnp.abs(z))) + jnp.maximum(z, 0.0)


def _sigmoid(z):
    return 1.0 / (1.0 + jnp.exp(-z))


_RT = 1024
_NT = 128
_GT = _NT * M


def _embed_body(af_ref, wemb_ref, wcat_ref, x_ref, ps_ref, pn_ref):
    x = jnp.dot(af_ref[...], wemb_ref[...], preferred_element_type=jnp.float32)
    x_ref[...] = x
    p = jnp.dot(x, wcat_ref[...], preferred_element_type=jnp.float32)
    ps_ref[...] = p[:, :F2]
    pn_ref[...] = p[:, F2:]


def _embed(atom_fea, W_emb, Wcat):
    return pl.pallas_call(
        _embed_body,
        grid=(N // _RT,),
        in_specs=[
            pl.BlockSpec((_RT, ORIG), lambda i: (i, 0)),
            pl.BlockSpec((ORIG, AF), lambda i: (0, 0)),
            pl.BlockSpec((AF, 2 * F2), lambda i: (0, 0)),
        ],
        out_specs=[
            pl.BlockSpec((_RT, AF), lambda i: (i, 0)),
            pl.BlockSpec((_RT, F2), lambda i: (i, 0)),
            pl.BlockSpec((_RT, F2), lambda i: (i, 0)),
        ],
        out_shape=[
            jax.ShapeDtypeStruct((N, AF), jnp.float32),
            jax.ShapeDtypeStruct((N, F2), jnp.float32),
            jax.ShapeDtypeStruct((N, F2), jnp.float32),
        ],
    )(atom_fea, W_emb, Wcat)


def _stats_body(an_ref, nbr_ref, ps_ref, wfe_ref, g_ref, su_ref, sq_ref):
    pe = jnp.dot(nbr_ref[...], wfe_ref[...], preferred_element_type=jnp.float32)
    ps = ps_ref[...]
    psr = jnp.broadcast_to(ps[:, None, :], (_NT, M, F2)).reshape(_GT, F2)
    g = pe + an_ref[...] + psr
    g_ref[...] = g.astype(jnp.bfloat16)

    @pl.when(pl.program_id(0) == 0)
    def _():
        su_ref[...] = jnp.zeros_like(su_ref)
        sq_ref[...] = jnp.zeros_like(sq_ref)

    su_ref[...] += jnp.sum(g, axis=0, keepdims=True)
    sq_ref[...] += jnp.sum(g * g, axis=0, keepdims=True)


def _stats(an, nbr_flat, ps, Wfe):
    return pl.pallas_call(
        _stats_body,
        grid=(N // _NT,),
        in_specs=[
            pl.BlockSpec((_GT, F2), lambda i: (i, 0)),
            pl.BlockSpec((_GT, NBR), lambda i: (i, 0)),
            pl.BlockSpec((_NT, F2), lambda i: (i, 0)),
            pl.BlockSpec((NBR, F2), lambda i: (0, 0)),
        ],
        out_specs=[
            pl.BlockSpec((_GT, F2), lambda i: (i, 0)),
            pl.BlockSpec((1, F2), lambda i: (0, 0)),
            pl.BlockSpec((1, F2), lambda i: (0, 0)),
        ],
        out_shape=[
            jax.ShapeDtypeStruct((K, F2), jnp.bfloat16),
            jax.ShapeDtypeStruct((1, F2), jnp.float32),
            jax.ShapeDtypeStruct((1, F2), jnp.float32),
        ],
    )(an, nbr_flat, ps, Wfe)


def _act_body(g_ref, su_ref, sq_ref, g1_ref, bt1_ref, s_ref, ssu_ref, ssq_ref):
    mean = su_ref[...] / K
    var = sq_ref[...] / K - mean * mean
    a = g1_ref[...] * lax.rsqrt(var + 1e-5)
    c = bt1_ref[...] - mean * a
    g = g_ref[...].astype(jnp.float32)
    gh = g * a + c
    fl = _sigmoid(gh[:, :AF])
    co = _softplus(gh[:, AF:])
    s = jnp.sum((fl * co).reshape(_NT, M, AF), axis=1)
    s_ref[...] = s

    @pl.when(pl.program_id(0) == 0)
    def _():
        ssu_ref[...] = jnp.zeros_like(ssu_ref)
        ssq_ref[...] = jnp.zeros_like(ssq_ref)

    ssu_ref[...] += jnp.sum(s, axis=0, keepdims=True)
    ssq_ref[...] += jnp.sum(s * s, axis=0, keepdims=True)


def _act(g, su, sq, g1, bt1):
    return pl.pallas_call(
        _act_body,
        grid=(N // _NT,),
        in_specs=[
            pl.BlockSpec((_GT, F2), lambda i: (i, 0)),
            pl.BlockSpec((1, F2), lambda i: (0, 0)),
            pl.BlockSpec((1, F2), lambda i: (0, 0)),
            pl.BlockSpec((1, F2), lambda i: (0, 0)),
            pl.BlockSpec((1, F2), lambda i: (0, 0)),
        ],
        out_specs=[
            pl.BlockSpec((_NT, AF), lambda i: (i, 0)),
            pl.BlockSpec((1, AF), lambda i: (0, 0)),
            pl.BlockSpec((1, AF), lambda i: (0, 0)),
        ],
        out_shape=[
            jax.ShapeDtypeStruct((N, AF), jnp.float32),
            jax.ShapeDtypeStruct((1, AF), jnp.float32),
            jax.ShapeDtypeStruct((1, AF), jnp.float32),
        ],
    )(g, su, sq, g1, bt1)


def _bn2_update(x_ref, s_ref, ssu_ref, ssq_ref, g2_ref, bt2_ref):
    m2 = ssu_ref[...] / N
    v2 = ssq_ref[...] / N - m2 * m2
    a2 = g2_ref[...] * lax.rsqrt(v2 + 1e-5)
    c2 = bt2_ref[...] - m2 * a2
    return _softplus(x_ref[...] + s_ref[...] * a2 + c2)


def _update_body(x_ref, s_ref, ssu_ref, ssq_ref, g2_ref, bt2_ref, wcat_ref,
                 xo_ref, ps_ref, pn_ref):
    xn = _bn2_update(x_ref, s_ref, ssu_ref, ssq_ref, g2_ref, bt2_ref)
    xo_ref[...] = xn
    p = jnp.dot(xn, wcat_ref[...], preferred_element_type=jnp.float32)
    ps_ref[...] = p[:, :F2]
    pn_ref[...] = p[:, F2:]


def _update(x, s, ssu, ssq, g2, bt2, Wcat):
    return pl.pallas_call(
        _update_body,
        grid=(N // _RT,),
        in_specs=[
            pl.BlockSpec((_RT, AF), lambda i: (i, 0)),
            pl.BlockSpec((_RT, AF), lambda i: (i, 0)),
            pl.BlockSpec((1, AF), lambda i: (0, 0)),
            pl.BlockSpec((1, AF), lambda i: (0, 0)),
            pl.BlockSpec((1, AF), lambda i: (0, 0)),
            pl.BlockSpec((1, AF), lambda i: (0, 0)),
            pl.BlockSpec((AF, 2 * F2), lambda i: (0, 0)),
        ],
        out_specs=[
            pl.BlockSpec((_RT, AF), lambda i: (i, 0)),
            pl.BlockSpec((_RT, F2), lambda i: (i, 0)),
            pl.BlockSpec((_RT, F2), lambda i: (i, 0)),
        ],
        out_shape=[
            jax.ShapeDtypeStruct((N, AF), jnp.float32),
            jax.ShapeDtypeStruct((N, F2), jnp.float32),
            jax.ShapeDtypeStruct((N, F2), jnp.float32),
        ],
    )(x, s, ssu, ssq, g2, bt2, Wcat)


def _update_last_body(x_ref, s_ref, ssu_ref, ssq_ref, g2_ref, bt2_ref, xo_ref):
    xo_ref[...] = _bn2_update(x_ref, s_ref, ssu_ref, ssq_ref, g2_ref, bt2_ref)


def _update_last(x, s, ssu, ssq, g2, bt2):
    return pl.pallas_call(
        _update_last_body,
        grid=(N // _RT,),
        in_specs=[
            pl.BlockSpec((_RT, AF), lambda i: (i, 0)),
            pl.BlockSpec((_RT, AF), lambda i: (i, 0)),
            pl.BlockSpec((1, AF), lambda i: (0, 0)),
            pl.BlockSpec((1, AF), lambda i: (0, 0)),
            pl.BlockSpec((1, AF), lambda i: (0, 0)),
            pl.BlockSpec((1, AF), lambda i: (0, 0)),
        ],
        out_specs=pl.BlockSpec((_RT, AF), lambda i: (i, 0)),
        out_shape=jax.ShapeDtypeStruct((N, AF), jnp.float32),
    )(x, s, ssu, ssq, g2, bt2)


def _final_body(x_ref, w2c_ref, b2_ref, waf_ref, baf_ref,
                ep_ref, af_ref, z_ref, n_ref):
    x = x_ref[...]
    nrm = jnp.sqrt(jnp.sum(x * x, axis=1, keepdims=True))
    nd = x / jnp.maximum(nrm, 1e-12)
    n_ref[...] = nd[None]
    z_ref[...] = jnp.mean(nd, axis=0, keepdims=True)[None]
    af_ref[...] = jnp.dot(nd, waf_ref[...], preferred_element_type=jnp.float32) + baf_ref[...]
    tmp = jnp.dot(nd, w2c_ref[...], preferred_element_type=jnp.float32)
    tmp_r = jnp.concatenate([tmp[:, j * AF:(j + 1) * AF] for j in range(6)], axis=0)
    es_all = lax.dot_general(tmp_r, nd, (((1,), (1,)), ((), ())),
                             preferred_element_type=jnp.float32)
    es = [es_all[j * A:(j + 1) * A, :] + b2_ref[0, j] for j in range(6)]
    mx = es[0]
    for j in range(1, 6):
        mx = jnp.maximum(mx, es[j])
    se = jnp.exp(es[0] - mx)
    for j in range(1, 6):
        se += jnp.exp(es[j] - mx)
    off = mx + jnp.log(se)
    ep_ref[...] = jnp.concatenate([es[j] - off for j in range(6)], axis=0)[None]


def _final(x, W2c, b2, W_af, b_af):
    return pl.pallas_call(
        _final_body,
        grid=(B,),
        in_specs=[
            pl.BlockSpec((A, AF), lambda i: (i, 0)),
            pl.BlockSpec((AF, 6 * AF), lambda i: (0, 0)),
            pl.BlockSpec((1, 6), lambda i: (0, 0)),
            pl.BlockSpec((AF, ORIG), lambda i: (0, 0)),
            pl.BlockSpec((1, ORIG), lambda i: (0, 0)),
        ],
        out_specs=[
            pl.BlockSpec((1, 6 * A, A), lambda i: (i, 0, 0)),
            pl.BlockSpec((A, ORIG), lambda i: (i, 0)),
            pl.BlockSpec((1, 1, AF), lambda i: (i, 0, 0)),
            pl.BlockSpec((1, A, AF), lambda i: (i, 0, 0)),
        ],
        out_shape=[
            jax.ShapeDtypeStruct((B, 6 * A, A), jnp.float32),
            jax.ShapeDtypeStruct((N, ORIG), jnp.float32),
            jax.ShapeDtypeStruct((B, 1, AF), jnp.float32),
            jax.ShapeDtypeStruct((B, A, AF), jnp.float32),
        ],
    )(x, W2c, b2, W_af, b_af)


def kernel(atom_fea, nbr_fea, nbr_fea_idx, crystal_atom_idx, cuda_flag, W_emb,
           Wf0, bf0, g1_0, bt1_0, g2_0, bt2_0,
           Wf1, bf1, g1_1, bt1_1, g2_1, bt2_1,
           Wf2, bf2, g1_2, bt1_2, g2_2, bt2_2,
           W_bil, b_bil, W_fc1, b_fc1, W_af, b_af):
    Wf = [Wf0, Wf1, Wf2]
    g1 = [g1_0[None], g1_1[None], g1_2[None]]
    bt1 = [bt1_0[None], bt1_1[None], bt1_2[None]]
    g2 = [g2_0[None], g2_1[None], g2_2[None]]
    bt2 = [bt2_0[None], bt2_1[None], bt2_2[None]]
    Wcat = [jnp.concatenate([w[:AF], w[AF:2 * AF]], axis=1) for w in Wf]
    Wfe = [w[2 * AF:].astype(jnp.bfloat16) for w in Wf]
    idx = nbr_fea_idx.reshape(-1).astype(jnp.int32)
    nbr_flat = nbr_fea.reshape(K, NBR).astype(jnp.bfloat16)

    x, ps, pn = _embed(atom_fea, W_emb, Wcat[0])
    for l in range(NC):
        an = _sc_gather(pn, idx)
        g, su, sq = _stats(an, nbr_flat, ps, Wfe[l])
        s, ssu, ssq = _act(g, su, sq, g1[l], bt1[l])
        if l + 1 < NC:
            x, ps, pn = _update(x, s, ssu, ssq, g2[l], bt2[l], Wcat[l + 1])
        else:
            x = _update_last(x, s, ssu, ssq, g2[l], bt2[l])

    W2 = jnp.einsum('kde,kj->jde', W_bil, W_fc1)
    W2c = jnp.concatenate([W2[j] for j in range(6)], axis=1)
    b2 = (b_bil @ W_fc1 + b_fc1)[None]
    epk, af, z, normed = _final(x, W2c, b2, W_af, b_af[None])
    ep = jnp.transpose(epk.reshape(B, 6, A, A), (0, 2, 3, 1)).reshape(-1, 6)
    return ep, af, z.reshape(B, AF), normed, x


# speedup vs baseline: 3.5686x; 1.0827x over previous
import functools

import jax
import jax.numpy as jnp
from jax import lax
from jax.experimental import pallas as pl
from jax.experimental.pallas import tpu as pltpu
from jax.experimental.pallas import tpu_sc as plsc

B, A, M = 64, 128, 12
ORIG, NBR, AF, NC = 92, 41, 64, 3
N = B * A
K = N * M
F2 = 2 * AF

_SC_CORES, _SC_SUBCORES = 2, 16
_NW = _SC_CORES * _SC_SUBCORES
_ROWS_PER_W = K // _NW
_CH = 128
_NCH = _ROWS_PER_W // _CH


def _sc_gather(table, idx):
    mesh = plsc.VectorSubcoreMesh(core_axis_name="c", subcore_axis_name="s")

    @functools.partial(
        pl.kernel, mesh=mesh,
        out_type=jax.ShapeDtypeStruct((K, F2), jnp.float32),
        scratch_types=[
            pltpu.VMEM((_ROWS_PER_W,), jnp.int32),
            pltpu.VMEM((_CH, F2), jnp.float32),
            pltpu.VMEM((_CH, F2), jnp.float32),
            pltpu.SemaphoreType.DMA,
            pltpu.SemaphoreType.DMA,
        ],
    )
    def gk(table_hbm, idx_hbm, out_hbm, idx_all, rows0, rows1, sem0, sem1):
        wid = lax.axis_index("s") * _SC_CORES + lax.axis_index("c")
        base_w = pl.multiple_of(wid * _ROWS_PER_W, _ROWS_PER_W)
        rows = (rows0, rows1)
        sems = (sem0, sem1)
        pltpu.sync_copy(idx_hbm.at[pl.ds(base_w, _ROWS_PER_W)], idx_all)
        for b in range(2):
            pltpu.async_copy(
                table_hbm.at[idx_all.at[pl.ds(b * _CH, _CH)]], rows[b], sems[b])

        def body(it, carry):
            for b in range(2):
                j = 2 * it + b
                pltpu.make_async_copy(
                    table_hbm.at[idx_all.at[pl.ds(0, _CH)]], rows[b], sems[b]).wait()
                pltpu.sync_copy(
                    rows[b], out_hbm.at[pl.ds(base_w + j * _CH, _CH)])
                j2 = j + 2

                @pl.when(j2 < _NCH)
                def _():
                    pltpu.async_copy(
                        table_hbm.at[idx_all.at[pl.ds(j2 * _CH, _CH)]],
                        rows[b], sems[b])
            return carry

        lax.fori_loop(0, _NCH // 2, body, 0)

    return gk(table, idx)


def _softplus(z):
    return jnp.log(1.0 + jnp.exp(-jnp.abs(z))) + jnp.maximum(z, 0.0)


def _sigmoid(z):
    return 1.0 / (1.0 + jnp.exp(-z))


_RT = 1024
_NT = 128
_GT = _NT * M


def _embed_body(af_ref, wemb_ref, wcat_ref, x_ref, ps_ref, pn_ref):
    x = jnp.dot(af_ref[...], wemb_ref[...], preferred_element_type=jnp.float32)
    x_ref[...] = x
    p = jnp.dot(x, wcat_ref[...], preferred_element_type=jnp.float32)
    ps_ref[...] = p[:, :F2]
    pn_ref[...] = p[:, F2:]


def _embed(atom_fea, W_emb, Wcat):
    return pl.pallas_call(
        _embed_body,
        grid=(N // _RT,),
        in_specs=[
            pl.BlockSpec((_RT, ORIG), lambda i: (i, 0)),
            pl.BlockSpec((ORIG, AF), lambda i: (0, 0)),
            pl.BlockSpec((AF, 2 * F2), lambda i: (0, 0)),
        ],
        out_specs=[
            pl.BlockSpec((_RT, AF), lambda i: (i, 0)),
            pl.BlockSpec((_RT, F2), lambda i: (i, 0)),
            pl.BlockSpec((_RT, F2), lambda i: (i, 0)),
        ],
        out_shape=[
            jax.ShapeDtypeStruct((N, AF), jnp.float32),
            jax.ShapeDtypeStruct((N, F2), jnp.float32),
            jax.ShapeDtypeStruct((N, F2), jnp.float32),
        ],
    )(atom_fea, W_emb, Wcat)


def _stats_body(an_ref, nbr_ref, ps_ref, wfe_ref, g_ref, su_ref, sq_ref):
    pe = jnp.dot(nbr_ref[...], wfe_ref[...], preferred_element_type=jnp.float32)
    ps = ps_ref[...]
    psr = jnp.broadcast_to(ps[:, None, :], (_NT, M, F2)).reshape(_GT, F2)
    g = pe + an_ref[...] + psr
    g_ref[...] = g.astype(jnp.bfloat16)

    @pl.when(pl.program_id(0) == 0)
    def _():
        su_ref[...] = jnp.zeros_like(su_ref)
        sq_ref[...] = jnp.zeros_like(sq_ref)

    su_ref[...] += jnp.sum(g, axis=0, keepdims=True)
    sq_ref[...] += jnp.sum(g * g, axis=0, keepdims=True)


def _stats(an, nbr_flat, ps, Wfe):
    return pl.pallas_call(
        _stats_body,
        grid=(N // _NT,),
        in_specs=[
            pl.BlockSpec((_GT, F2), lambda i: (i, 0)),
            pl.BlockSpec((_GT, NBR), lambda i: (i, 0)),
            pl.BlockSpec((_NT, F2), lambda i: (i, 0)),
            pl.BlockSpec((NBR, F2), lambda i: (0, 0)),
        ],
        out_specs=[
            pl.BlockSpec((_GT, F2), lambda i: (i, 0)),
            pl.BlockSpec((1, F2), lambda i: (0, 0)),
            pl.BlockSpec((1, F2), lambda i: (0, 0)),
        ],
        out_shape=[
            jax.ShapeDtypeStruct((K, F2), jnp.bfloat16),
            jax.ShapeDtypeStruct((1, F2), jnp.float32),
            jax.ShapeDtypeStruct((1, F2), jnp.float32),
        ],
    )(an, nbr_flat, ps, Wfe)


def _act_body(g_ref, su_ref, sq_ref, g1_ref, bt1_ref, s_ref, ssu_ref, ssq_ref):
    mean = su_ref[...] / K
    var = sq_ref[...] / K - mean * mean
    a = g1_ref[...] * lax.rsqrt(var + 1e-5)
    c = bt1_ref[...] - mean * a
    g = g_ref[...].astype(jnp.float32)
    gh = g * a + c
    fl = _sigmoid(gh[:, :AF])
    co = _softplus(gh[:, AF:])
    s = jnp.sum((fl * co).reshape(_NT, M, AF), axis=1)
    s_ref[...] = s

    @pl.when(pl.program_id(0) == 0)
    def _():
        ssu_ref[...] = jnp.zeros_like(ssu_ref)
        ssq_ref[...] = jnp.zeros_like(ssq_ref)

    ssu_ref[...] += jnp.sum(s, axis=0, keepdims=True)
    ssq_ref[...] += jnp.sum(s * s, axis=0, keepdims=True)


def _act(g, su, sq, g1, bt1):
    return pl.pallas_call(
        _act_body,
        grid=(N // _NT,),
        in_specs=[
            pl.BlockSpec((_GT, F2), lambda i: (i, 0)),
            pl.BlockSpec((1, F2), lambda i: (0, 0)),
            pl.BlockSpec((1, F2), lambda i: (0, 0)),
            pl.BlockSpec((1, F2), lambda i: (0, 0)),
            pl.BlockSpec((1, F2), lambda i: (0, 0)),
        ],
        out_specs=[
            pl.BlockSpec((_NT, AF), lambda i: (i, 0)),
            pl.BlockSpec((1, AF), lambda i: (0, 0)),
            pl.BlockSpec((1, AF), lambda i: (0, 0)),
        ],
        out_shape=[
            jax.ShapeDtypeStruct((N, AF), jnp.float32),
            jax.ShapeDtypeStruct((1, AF), jnp.float32),
            jax.ShapeDtypeStruct((1, AF), jnp.float32),
        ],
    )(g, su, sq, g1, bt1)


def _bn2_update(x_ref, s_ref, ssu_ref, ssq_ref, g2_ref, bt2_ref):
    m2 = ssu_ref[...] / N
    v2 = ssq_ref[...] / N - m2 * m2
    a2 = g2_ref[...] * lax.rsqrt(v2 + 1e-5)
    c2 = bt2_ref[...] - m2 * a2
    return _softplus(x_ref[...] + s_ref[...] * a2 + c2)


def _update_body(x_ref, s_ref, ssu_ref, ssq_ref, g2_ref, bt2_ref, wcat_ref,
                 xo_ref, ps_ref, pn_ref):
    xn = _bn2_update(x_ref, s_ref, ssu_ref, ssq_ref, g2_ref, bt2_ref)
    xo_ref[...] = xn
    p = jnp.dot(xn, wcat_ref[...], preferred_element_type=jnp.float32)
    ps_ref[...] = p[:, :F2]
    pn_ref[...] = p[:, F2:]


def _update(x, s, ssu, ssq, g2, bt2, Wcat):
    return pl.pallas_call(
        _update_body,
        grid=(N // _RT,),
        in_specs=[
            pl.BlockSpec((_RT, AF), lambda i: (i, 0)),
            pl.BlockSpec((_RT, AF), lambda i: (i, 0)),
            pl.BlockSpec((1, AF), lambda i: (0, 0)),
            pl.BlockSpec((1, AF), lambda i: (0, 0)),
            pl.BlockSpec((1, AF), lambda i: (0, 0)),
            pl.BlockSpec((1, AF), lambda i: (0, 0)),
            pl.BlockSpec((AF, 2 * F2), lambda i: (0, 0)),
        ],
        out_specs=[
            pl.BlockSpec((_RT, AF), lambda i: (i, 0)),
            pl.BlockSpec((_RT, F2), lambda i: (i, 0)),
            pl.BlockSpec((_RT, F2), lambda i: (i, 0)),
        ],
        out_shape=[
            jax.ShapeDtypeStruct((N, AF), jnp.float32),
            jax.ShapeDtypeStruct((N, F2), jnp.float32),
            jax.ShapeDtypeStruct((N, F2), jnp.float32),
        ],
    )(x, s, ssu, ssq, g2, bt2, Wcat)


def _update_last_body(x_ref, s_ref, ssu_ref, ssq_ref, g2_ref, bt2_ref, xo_ref):
    xo_ref[...] = _bn2_update(x_ref, s_ref, ssu_ref, ssq_ref, g2_ref, bt2_ref)


def _update_last(x, s, ssu, ssq, g2, bt2):
    return pl.pallas_call(
        _update_last_body,
        grid=(N // _RT,),
        in_specs=[
            pl.BlockSpec((_RT, AF), lambda i: (i, 0)),
            pl.BlockSpec((_RT, AF), lambda i: (i, 0)),
            pl.BlockSpec((1, AF), lambda i: (0, 0)),
            pl.BlockSpec((1, AF), lambda i: (0, 0)),
            pl.BlockSpec((1, AF), lambda i: (0, 0)),
            pl.BlockSpec((1, AF), lambda i: (0, 0)),
        ],
        out_specs=pl.BlockSpec((_RT, AF), lambda i: (i, 0)),
        out_shape=jax.ShapeDtypeStruct((N, AF), jnp.float32),
    )(x, s, ssu, ssq, g2, bt2)


def _final_body(x_ref, w2c_ref, b2_ref, waf_ref, baf_ref,
                ep_ref, af_ref, z_ref, n_ref):
    x = x_ref[...]
    nrm = jnp.sqrt(jnp.sum(x * x, axis=1, keepdims=True))
    nd = x / jnp.maximum(nrm, 1e-12)
    n_ref[...] = nd[None]
    z_ref[...] = jnp.mean(nd, axis=0, keepdims=True)[None]
    af_ref[...] = jnp.dot(nd, waf_ref[...], preferred_element_type=jnp.float32) + baf_ref[...]
    tmp = jnp.dot(nd, w2c_ref[...], preferred_element_type=jnp.float32)
    tmp_r = jnp.concatenate([tmp[:, j * AF:(j + 1) * AF] for j in range(6)], axis=0)
    es_all = lax.dot_general(tmp_r, nd, (((1,), (1,)), ((), ())),
                             preferred_element_type=jnp.float32)
    es = [es_all[j * A:(j + 1) * A, :] + b2_ref[0, j] for j in range(6)]
    mx = es[0]
    for j in range(1, 6):
        mx = jnp.maximum(mx, es[j])
    se = jnp.exp(es[0] - mx)
    for j in range(1, 6):
        se += jnp.exp(es[j] - mx)
    off = mx + jnp.log(se)
    ep_ref[...] = jnp.concatenate([es[j] - off for j in range(6)], axis=0)[None]


def _final(x, W2c, b2, W_af, b_af):
    return pl.pallas_call(
        _final_body,
        grid=(B,),
        in_specs=[
            pl.BlockSpec((A, AF), lambda i: (i, 0)),
            pl.BlockSpec((AF, 6 * AF), lambda i: (0, 0)),
            pl.BlockSpec((1, 6), lambda i: (0, 0)),
            pl.BlockSpec((AF, ORIG), lambda i: (0, 0)),
            pl.BlockSpec((1, ORIG), lambda i: (0, 0)),
        ],
        out_specs=[
            pl.BlockSpec((1, 6 * A, A), lambda i: (i, 0, 0)),
            pl.BlockSpec((A, ORIG), lambda i: (i, 0)),
            pl.BlockSpec((1, 1, AF), lambda i: (i, 0, 0)),
            pl.BlockSpec((1, A, AF), lambda i: (i, 0, 0)),
        ],
        out_shape=[
            jax.ShapeDtypeStruct((B, 6 * A, A), jnp.float32),
            jax.ShapeDtypeStruct((N, ORIG), jnp.float32),
            jax.ShapeDtypeStruct((B, 1, AF), jnp.float32),
            jax.ShapeDtypeStruct((B, A, AF), jnp.float32),
        ],
    )(x, W2c, b2, W_af, b_af)


def kernel(atom_fea, nbr_fea, nbr_fea_idx, crystal_atom_idx, cuda_flag, W_emb,
           Wf0, bf0, g1_0, bt1_0, g2_0, bt2_0,
           Wf1, bf1, g1_1, bt1_1, g2_1, bt2_1,
           Wf2, bf2, g1_2, bt1_2, g2_2, bt2_2,
           W_bil, b_bil, W_fc1, b_fc1, W_af, b_af):
    Wf = [Wf0, Wf1, Wf2]
    g1 = [g1_0[None], g1_1[None], g1_2[None]]
    bt1 = [bt1_0[None], bt1_1[None], bt1_2[None]]
    g2 = [g2_0[None], g2_1[None], g2_2[None]]
    bt2 = [bt2_0[None], bt2_1[None], bt2_2[None]]
    Wcat = [jnp.concatenate([w[:AF], w[AF:2 * AF]], axis=1) for w in Wf]
    Wfe = [w[2 * AF:].astype(jnp.bfloat16) for w in Wf]
    idx = nbr_fea_idx.reshape(-1).astype(jnp.int32)
    nbr_flat = nbr_fea.reshape(K, NBR).astype(jnp.bfloat16)

    x, ps, pn = _embed(atom_fea, W_emb, Wcat[0])
    for l in range(NC):
        an = _sc_gather(pn, idx)
        g, su, sq = _stats(an, nbr_flat, ps, Wfe[l])
        s, ssu, ssq = _act(g, su, sq, g1[l], bt1[l])
        if l + 1 < NC:
            x, ps, pn = _update(x, s, ssu, ssq, g2[l], bt2[l], Wcat[l + 1])
        else:
            x = _update_last(x, s, ssu, ssq, g2[l], bt2[l])

    W2 = jnp.einsum('kde,kj->jde', W_bil, W_fc1)
    W2c = jnp.concatenate([W2[j] for j in range(6)], axis=1)
    b2 = (b_bil @ W_fc1 + b_fc1)[None]
    epk, af, z, normed = _final(x, W2c, b2, W_af, b_af[None])
    ep = jnp.transpose(epk.reshape(B, 6, A, A), (0, 2, 3, 1)).reshape(-1, 6)
    return ep, af, z.reshape(B, AF), normed, x


# speedup vs baseline: 4.1070x; 1.1509x over previous
import functools

import jax
import jax.numpy as jnp
from jax import lax
from jax.experimental import pallas as pl
from jax.experimental.pallas import tpu as pltpu
from jax.experimental.pallas import tpu_sc as plsc

B, A, M = 64, 128, 12
ORIG, NBR, AF, NC = 92, 41, 64, 3
N = B * A
K = N * M
F2 = 2 * AF

_SC_CORES, _SC_SUBCORES = 2, 16
_NW = _SC_CORES * _SC_SUBCORES
_ROWS_PER_W = K // _NW
_CH = 128
_NCH = _ROWS_PER_W // _CH


def _sc_gather(table, idx):
    mesh = plsc.VectorSubcoreMesh(core_axis_name="c", subcore_axis_name="s")

    @functools.partial(
        pl.kernel, mesh=mesh,
        out_type=jax.ShapeDtypeStruct((K, F2), jnp.float32),
        scratch_types=[
            pltpu.VMEM((_ROWS_PER_W,), jnp.int32),
            pltpu.VMEM((_CH, F2), jnp.float32),
            pltpu.VMEM((_CH, F2), jnp.float32),
            pltpu.SemaphoreType.DMA,
            pltpu.SemaphoreType.DMA,
        ],
    )
    def gk(table_hbm, idx_hbm, out_hbm, idx_all, rows0, rows1, sem0, sem1):
        wid = lax.axis_index("s") * _SC_CORES + lax.axis_index("c")
        base_w = pl.multiple_of(wid * _ROWS_PER_W, _ROWS_PER_W)
        rows = (rows0, rows1)
        sems = (sem0, sem1)
        pltpu.sync_copy(idx_hbm.at[pl.ds(base_w, _ROWS_PER_W)], idx_all)
        for b in range(2):
            pltpu.async_copy(
                table_hbm.at[idx_all.at[pl.ds(b * _CH, _CH)]], rows[b], sems[b])

        def body(it, carry):
            for b in range(2):
                j = 2 * it + b
                pltpu.make_async_copy(
                    table_hbm.at[idx_all.at[pl.ds(0, _CH)]], rows[b], sems[b]).wait()
                pltpu.sync_copy(
                    rows[b], out_hbm.at[pl.ds(base_w + j * _CH, _CH)])
                j2 = j + 2

                @pl.when(j2 < _NCH)
                def _():
                    pltpu.async_copy(
                        table_hbm.at[idx_all.at[pl.ds(j2 * _CH, _CH)]],
                        rows[b], sems[b])
            return carry

        lax.fori_loop(0, _NCH // 2, body, 0)

    return gk(table, idx)


def _softplus(z):
    return jnp.log(1.0 + jnp.exp(-jnp.abs(z))) + jnp.maximum(z, 0.0)


def _sigmoid(z):
    return 1.0 / (1.0 + jnp.exp(-z))


_RT = 1024
_NT = 256
_GT = _NT * M


def _embed_body(af_ref, wemb_ref, wcat_ref, x_ref, ps_ref, pn_ref):
    x = jnp.dot(af_ref[...], wemb_ref[...], preferred_element_type=jnp.float32)
    x_ref[...] = x
    p = jnp.dot(x, wcat_ref[...], preferred_element_type=jnp.float32)
    ps_ref[...] = p[:, :F2]
    pn_ref[...] = p[:, F2:]


def _embed(atom_fea, W_emb, Wcat):
    return pl.pallas_call(
        _embed_body,
        grid=(N // _RT,),
        in_specs=[
            pl.BlockSpec((_RT, ORIG), lambda i: (i, 0)),
            pl.BlockSpec((ORIG, AF), lambda i: (0, 0)),
            pl.BlockSpec((AF, 2 * F2), lambda i: (0, 0)),
        ],
        out_specs=[
            pl.BlockSpec((_RT, AF), lambda i: (i, 0)),
            pl.BlockSpec((_RT, F2), lambda i: (i, 0)),
            pl.BlockSpec((_RT, F2), lambda i: (i, 0)),
        ],
        out_shape=[
            jax.ShapeDtypeStruct((N, AF), jnp.float32),
            jax.ShapeDtypeStruct((N, F2), jnp.float32),
            jax.ShapeDtypeStruct((N, F2), jnp.float32),
        ],
    )(atom_fea, W_emb, Wcat)


def _stats_body(an_ref, nbr_ref, ps_ref, wfe_ref, g_ref, su_ref, sq_ref):
    pe = jnp.dot(nbr_ref[...], wfe_ref[...], preferred_element_type=jnp.float32)
    ps = ps_ref[...]
    psr = jnp.broadcast_to(ps[:, None, :], (_NT, M, F2)).reshape(_GT, F2)
    g = pe + an_ref[...] + psr
    g_ref[...] = g.astype(jnp.bfloat16)

    @pl.when(pl.program_id(0) == 0)
    def _():
        su_ref[...] = jnp.zeros_like(su_ref)
        sq_ref[...] = jnp.zeros_like(sq_ref)

    su_ref[...] += jnp.sum(g, axis=0, keepdims=True)
    sq_ref[...] += jnp.sum(g * g, axis=0, keepdims=True)


def _stats(an, nbr_flat, ps, Wfe):
    return pl.pallas_call(
        _stats_body,
        grid=(N // _NT,),
        in_specs=[
            pl.BlockSpec((_GT, F2), lambda i: (i, 0)),
            pl.BlockSpec((_GT, NBR), lambda i: (i, 0)),
            pl.BlockSpec((_NT, F2), lambda i: (i, 0)),
            pl.BlockSpec((NBR, F2), lambda i: (0, 0)),
        ],
        out_specs=[
            pl.BlockSpec((_GT, F2), lambda i: (i, 0)),
            pl.BlockSpec((1, F2), lambda i: (0, 0)),
            pl.BlockSpec((1, F2), lambda i: (0, 0)),
        ],
        out_shape=[
            jax.ShapeDtypeStruct((K, F2), jnp.bfloat16),
            jax.ShapeDtypeStruct((1, F2), jnp.float32),
            jax.ShapeDtypeStruct((1, F2), jnp.float32),
        ],
    )(an, nbr_flat, ps, Wfe)


def _act_body(g_ref, su_ref, sq_ref, g1_ref, bt1_ref, s_ref, ssu_ref, ssq_ref):
    mean = su_ref[...] / K
    var = sq_ref[...] / K - mean * mean
    a = g1_ref[...] * lax.rsqrt(var + 1e-5)
    c = bt1_ref[...] - mean * a
    g = g_ref[...].astype(jnp.float32)
    gh = g * a + c
    fl = _sigmoid(gh[:, :AF])
    co = _softplus(gh[:, AF:])
    s = jnp.sum((fl * co).reshape(_NT, M, AF), axis=1)
    s_ref[...] = s

    @pl.when(pl.program_id(0) == 0)
    def _():
        ssu_ref[...] = jnp.zeros_like(ssu_ref)
        ssq_ref[...] = jnp.zeros_like(ssq_ref)

    ssu_ref[...] += jnp.sum(s, axis=0, keepdims=True)
    ssq_ref[...] += jnp.sum(s * s, axis=0, keepdims=True)


def _act(g, su, sq, g1, bt1):
    return pl.pallas_call(
        _act_body,
        grid=(N // _NT,),
        in_specs=[
            pl.BlockSpec((_GT, F2), lambda i: (i, 0)),
            pl.BlockSpec((1, F2), lambda i: (0, 0)),
            pl.BlockSpec((1, F2), lambda i: (0, 0)),
            pl.BlockSpec((1, F2), lambda i: (0, 0)),
            pl.BlockSpec((1, F2), lambda i: (0, 0)),
        ],
        out_specs=[
            pl.BlockSpec((_NT, AF), lambda i: (i, 0)),
            pl.BlockSpec((1, AF), lambda i: (0, 0)),
            pl.BlockSpec((1, AF), lambda i: (0, 0)),
        ],
        out_shape=[
            jax.ShapeDtypeStruct((N, AF), jnp.float32),
            jax.ShapeDtypeStruct((1, AF), jnp.float32),
            jax.ShapeDtypeStruct((1, AF), jnp.float32),
        ],
    )(g, su, sq, g1, bt1)


def _bn2_update(x_ref, s_ref, ssu_ref, ssq_ref, g2_ref, bt2_ref):
    m2 = ssu_ref[...] / N
    v2 = ssq_ref[...] / N - m2 * m2
    a2 = g2_ref[...] * lax.rsqrt(v2 + 1e-5)
    c2 = bt2_ref[...] - m2 * a2
    return _softplus(x_ref[...] + s_ref[...] * a2 + c2)


def _update_body(x_ref, s_ref, ssu_ref, ssq_ref, g2_ref, bt2_ref, wcat_ref,
                 xo_ref, ps_ref, pn_ref):
    xn = _bn2_update(x_ref, s_ref, ssu_ref, ssq_ref, g2_ref, bt2_ref)
    xo_ref[...] = xn
    p = jnp.dot(xn, wcat_ref[...], preferred_element_type=jnp.float32)
    ps_ref[...] = p[:, :F2]
    pn_ref[...] = p[:, F2:]


def _update(x, s, ssu, ssq, g2, bt2, Wcat):
    return pl.pallas_call(
        _update_body,
        grid=(N // _RT,),
        in_specs=[
            pl.BlockSpec((_RT, AF), lambda i: (i, 0)),
            pl.BlockSpec((_RT, AF), lambda i: (i, 0)),
            pl.BlockSpec((1, AF), lambda i: (0, 0)),
            pl.BlockSpec((1, AF), lambda i: (0, 0)),
            pl.BlockSpec((1, AF), lambda i: (0, 0)),
            pl.BlockSpec((1, AF), lambda i: (0, 0)),
            pl.BlockSpec((AF, 2 * F2), lambda i: (0, 0)),
        ],
        out_specs=[
            pl.BlockSpec((_RT, AF), lambda i: (i, 0)),
            pl.BlockSpec((_RT, F2), lambda i: (i, 0)),
            pl.BlockSpec((_RT, F2), lambda i: (i, 0)),
        ],
        out_shape=[
            jax.ShapeDtypeStruct((N, AF), jnp.float32),
            jax.ShapeDtypeStruct((N, F2), jnp.float32),
            jax.ShapeDtypeStruct((N, F2), jnp.float32),
        ],
    )(x, s, ssu, ssq, g2, bt2, Wcat)


def _update_last_body(x_ref, s_ref, ssu_ref, ssq_ref, g2_ref, bt2_ref, xo_ref):
    xo_ref[...] = _bn2_update(x_ref, s_ref, ssu_ref, ssq_ref, g2_ref, bt2_ref)


def _update_last(x, s, ssu, ssq, g2, bt2):
    return pl.pallas_call(
        _update_last_body,
        grid=(N // _RT,),
        in_specs=[
            pl.BlockSpec((_RT, AF), lambda i: (i, 0)),
            pl.BlockSpec((_RT, AF), lambda i: (i, 0)),
            pl.BlockSpec((1, AF), lambda i: (0, 0)),
            pl.BlockSpec((1, AF), lambda i: (0, 0)),
            pl.BlockSpec((1, AF), lambda i: (0, 0)),
            pl.BlockSpec((1, AF), lambda i: (0, 0)),
        ],
        out_specs=pl.BlockSpec((_RT, AF), lambda i: (i, 0)),
        out_shape=jax.ShapeDtypeStruct((N, AF), jnp.float32),
    )(x, s, ssu, ssq, g2, bt2)


def _final_body(x_ref, w2c_ref, b2_ref, waf_ref, baf_ref,
                ep_ref, af_ref, z_ref, n_ref):
    x = x_ref[...]
    nrm = jnp.sqrt(jnp.sum(x * x, axis=1, keepdims=True))
    nd = x / jnp.maximum(nrm, 1e-12)
    n_ref[...] = nd[None]
    z_ref[...] = jnp.mean(nd, axis=0, keepdims=True)[None]
    af_ref[...] = jnp.dot(nd, waf_ref[...], preferred_element_type=jnp.float32) + baf_ref[...]
    tmp = jnp.dot(nd, w2c_ref[...], preferred_element_type=jnp.float32)
    tmp_r = jnp.concatenate([tmp[:, j * AF:(j + 1) * AF] for j in range(6)], axis=0)
    es_all = lax.dot_general(tmp_r, nd, (((1,), (1,)), ((), ())),
                             preferred_element_type=jnp.float32)
    es = [es_all[j * A:(j + 1) * A, :] + b2_ref[0, j] for j in range(6)]
    mx = es[0]
    for j in range(1, 6):
        mx = jnp.maximum(mx, es[j])
    se = jnp.exp(es[0] - mx)
    for j in range(1, 6):
        se += jnp.exp(es[j] - mx)
    off = mx + jnp.log(se)
    ep_ref[...] = jnp.concatenate([es[j] - off for j in range(6)], axis=0)[None]


def _final(x, W2c, b2, W_af, b_af):
    return pl.pallas_call(
        _final_body,
        grid=(B,),
        in_specs=[
            pl.BlockSpec((A, AF), lambda i: (i, 0)),
            pl.BlockSpec((AF, 6 * AF), lambda i: (0, 0)),
            pl.BlockSpec((1, 6), lambda i: (0, 0)),
            pl.BlockSpec((AF, ORIG), lambda i: (0, 0)),
            pl.BlockSpec((1, ORIG), lambda i: (0, 0)),
        ],
        out_specs=[
            pl.BlockSpec((1, 6 * A, A), lambda i: (i, 0, 0)),
            pl.BlockSpec((A, ORIG), lambda i: (i, 0)),
            pl.BlockSpec((1, 1, AF), lambda i: (i, 0, 0)),
            pl.BlockSpec((1, A, AF), lambda i: (i, 0, 0)),
        ],
        out_shape=[
            jax.ShapeDtypeStruct((B, 6 * A, A), jnp.float32),
            jax.ShapeDtypeStruct((N, ORIG), jnp.float32),
            jax.ShapeDtypeStruct((B, 1, AF), jnp.float32),
            jax.ShapeDtypeStruct((B, A, AF), jnp.float32),
        ],
    )(x, W2c, b2, W_af, b_af)


def kernel(atom_fea, nbr_fea, nbr_fea_idx, crystal_atom_idx, cuda_flag, W_emb,
           Wf0, bf0, g1_0, bt1_0, g2_0, bt2_0,
           Wf1, bf1, g1_1, bt1_1, g2_1, bt2_1,
           Wf2, bf2, g1_2, bt1_2, g2_2, bt2_2,
           W_bil, b_bil, W_fc1, b_fc1, W_af, b_af):
    Wf = [Wf0, Wf1, Wf2]
    g1 = [g1_0[None], g1_1[None], g1_2[None]]
    bt1 = [bt1_0[None], bt1_1[None], bt1_2[None]]
    g2 = [g2_0[None], g2_1[None], g2_2[None]]
    bt2 = [bt2_0[None], bt2_1[None], bt2_2[None]]
    Wcat = [jnp.concatenate([w[:AF], w[AF:2 * AF]], axis=1) for w in Wf]
    Wfe = [w[2 * AF:].astype(jnp.bfloat16) for w in Wf]
    idx = nbr_fea_idx.reshape(-1).astype(jnp.int32)
    nbr_flat = nbr_fea.reshape(K, NBR).astype(jnp.bfloat16)

    x, ps, pn = _embed(atom_fea, W_emb, Wcat[0])
    for l in range(NC):
        an = _sc_gather(pn, idx)
        g, su, sq = _stats(an, nbr_flat, ps, Wfe[l])
        s, ssu, ssq = _act(g, su, sq, g1[l], bt1[l])
        if l + 1 < NC:
            x, ps, pn = _update(x, s, ssu, ssq, g2[l], bt2[l], Wcat[l + 1])
        else:
            x = _update_last(x, s, ssu, ssq, g2[l], bt2[l])

    W2 = jnp.einsum('kde,kj->jde', W_bil, W_fc1)
    W2c = jnp.concatenate([W2[j] for j in range(6)], axis=1)
    b2 = (b_bil @ W_fc1 + b_fc1)[None]
    epk, af, z, normed = _final(x, W2c, b2, W_af, b_af[None])
    ep = jnp.transpose(epk.reshape(B, 6, A, A), (0, 2, 3, 1)).reshape(-1, 6)
    return ep, af, z.reshape(B, AF), normed, x


# speedup vs baseline: 4.3137x; 1.0503x over previous
import functools

import jax
import jax.numpy as jnp
from jax import lax
from jax.experimental import pallas as pl
from jax.experimental.pallas import tpu as pltpu
from jax.experimental.pallas import tpu_sc as plsc

B, A, M = 64, 128, 12
ORIG, NBR, AF, NC = 92, 41, 64, 3
N = B * A
K = N * M
F2 = 2 * AF

_SC_CORES, _SC_SUBCORES = 2, 16
_NW = _SC_CORES * _SC_SUBCORES
_ROWS_PER_W = K // _NW
_CH = 128
_NCH = _ROWS_PER_W // _CH


def _sc_gather(table, idx):
    mesh = plsc.VectorSubcoreMesh(core_axis_name="c", subcore_axis_name="s")

    @functools.partial(
        pl.kernel, mesh=mesh,
        out_type=jax.ShapeDtypeStruct((K, F2), jnp.float32),
        scratch_types=[
            pltpu.VMEM((_ROWS_PER_W,), jnp.int32),
            pltpu.VMEM((_CH, F2), jnp.float32),
            pltpu.VMEM((_CH, F2), jnp.float32),
            pltpu.SemaphoreType.DMA,
            pltpu.SemaphoreType.DMA,
        ],
    )
    def gk(table_hbm, idx_hbm, out_hbm, idx_all, rows0, rows1, sem0, sem1):
        wid = lax.axis_index("s") * _SC_CORES + lax.axis_index("c")
        base_w = pl.multiple_of(wid * _ROWS_PER_W, _ROWS_PER_W)
        rows = (rows0, rows1)
        sems = (sem0, sem1)
        pltpu.sync_copy(idx_hbm.at[pl.ds(base_w, _ROWS_PER_W)], idx_all)
        for b in range(2):
            pltpu.async_copy(
                table_hbm.at[idx_all.at[pl.ds(b * _CH, _CH)]], rows[b], sems[b])

        def body(it, carry):
            for b in range(2):
                j = 2 * it + b
                pltpu.make_async_copy(
                    table_hbm.at[idx_all.at[pl.ds(0, _CH)]], rows[b], sems[b]).wait()
                pltpu.sync_copy(
                    rows[b], out_hbm.at[pl.ds(base_w + j * _CH, _CH)])
                j2 = j + 2

                @pl.when(j2 < _NCH)
                def _():
                    pltpu.async_copy(
                        table_hbm.at[idx_all.at[pl.ds(j2 * _CH, _CH)]],
                        rows[b], sems[b])
            return carry

        lax.fori_loop(0, _NCH // 2, body, 0)

    return gk(table, idx)


def _softplus(z):
    return jnp.log(1.0 + jnp.exp(-jnp.abs(z))) + jnp.maximum(z, 0.0)


def _sigmoid(z):
    return 1.0 / (1.0 + jnp.exp(-z))


_RT = 1024
_NT = 512
_GT = _NT * M


def _embed_body(af_ref, wemb_ref, wcat_ref, x_ref, ps_ref, pn_ref):
    x = jnp.dot(af_ref[...], wemb_ref[...], preferred_element_type=jnp.float32)
    x_ref[...] = x
    p = jnp.dot(x, wcat_ref[...], preferred_element_type=jnp.float32)
    ps_ref[...] = p[:, :F2]
    pn_ref[...] = p[:, F2:]


def _embed(atom_fea, W_emb, Wcat):
    return pl.pallas_call(
        _embed_body,
        grid=(N // _RT,),
        in_specs=[
            pl.BlockSpec((_RT, ORIG), lambda i: (i, 0)),
            pl.BlockSpec((ORIG, AF), lambda i: (0, 0)),
            pl.BlockSpec((AF, 2 * F2), lambda i: (0, 0)),
        ],
        out_specs=[
            pl.BlockSpec((_RT, AF), lambda i: (i, 0)),
            pl.BlockSpec((_RT, F2), lambda i: (i, 0)),
            pl.BlockSpec((_RT, F2), lambda i: (i, 0)),
        ],
        out_shape=[
            jax.ShapeDtypeStruct((N, AF), jnp.float32),
            jax.ShapeDtypeStruct((N, F2), jnp.float32),
            jax.ShapeDtypeStruct((N, F2), jnp.float32),
        ],
    )(atom_fea, W_emb, Wcat)


def _stats_body(an_ref, nbr_ref, ps_ref, wfe_ref, g_ref, su_ref, sq_ref):
    pe = jnp.dot(nbr_ref[...], wfe_ref[...], preferred_element_type=jnp.float32)
    ps = ps_ref[...]
    psr = jnp.broadcast_to(ps[:, None, :], (_NT, M, F2)).reshape(_GT, F2)
    g = pe + an_ref[...] + psr
    g_ref[...] = g.astype(jnp.bfloat16)

    @pl.when(pl.program_id(0) == 0)
    def _():
        su_ref[...] = jnp.zeros_like(su_ref)
        sq_ref[...] = jnp.zeros_like(sq_ref)

    su_ref[...] += jnp.sum(g, axis=0, keepdims=True)
    sq_ref[...] += jnp.sum(g * g, axis=0, keepdims=True)


def _stats(an, nbr_flat, ps, Wfe):
    return pl.pallas_call(
        _stats_body,
        grid=(N // _NT,),
        in_specs=[
            pl.BlockSpec((_GT, F2), lambda i: (i, 0)),
            pl.BlockSpec((_GT, NBR), lambda i: (i, 0)),
            pl.BlockSpec((_NT, F2), lambda i: (i, 0)),
            pl.BlockSpec((NBR, F2), lambda i: (0, 0)),
        ],
        out_specs=[
            pl.BlockSpec((_GT, F2), lambda i: (i, 0)),
            pl.BlockSpec((1, F2), lambda i: (0, 0)),
            pl.BlockSpec((1, F2), lambda i: (0, 0)),
        ],
        out_shape=[
            jax.ShapeDtypeStruct((K, F2), jnp.bfloat16),
            jax.ShapeDtypeStruct((1, F2), jnp.float32),
            jax.ShapeDtypeStruct((1, F2), jnp.float32),
        ],
    )(an, nbr_flat, ps, Wfe)


def _act_body(g_ref, su_ref, sq_ref, g1_ref, bt1_ref, s_ref, ssu_ref, ssq_ref):
    mean = su_ref[...] / K
    var = sq_ref[...] / K - mean * mean
    a = g1_ref[...] * lax.rsqrt(var + 1e-5)
    c = bt1_ref[...] - mean * a
    g = g_ref[...].astype(jnp.float32)
    gh = g * a + c
    fl = _sigmoid(gh[:, :AF])
    co = _softplus(gh[:, AF:])
    s = jnp.sum((fl * co).reshape(_NT, M, AF), axis=1)
    s_ref[...] = s

    @pl.when(pl.program_id(0) == 0)
    def _():
        ssu_ref[...] = jnp.zeros_like(ssu_ref)
        ssq_ref[...] = jnp.zeros_like(ssq_ref)

    ssu_ref[...] += jnp.sum(s, axis=0, keepdims=True)
    ssq_ref[...] += jnp.sum(s * s, axis=0, keepdims=True)


def _act(g, su, sq, g1, bt1):
    return pl.pallas_call(
        _act_body,
        grid=(N // _NT,),
        in_specs=[
            pl.BlockSpec((_GT, F2), lambda i: (i, 0)),
            pl.BlockSpec((1, F2), lambda i: (0, 0)),
            pl.BlockSpec((1, F2), lambda i: (0, 0)),
            pl.BlockSpec((1, F2), lambda i: (0, 0)),
            pl.BlockSpec((1, F2), lambda i: (0, 0)),
        ],
        out_specs=[
            pl.BlockSpec((_NT, AF), lambda i: (i, 0)),
            pl.BlockSpec((1, AF), lambda i: (0, 0)),
            pl.BlockSpec((1, AF), lambda i: (0, 0)),
        ],
        out_shape=[
            jax.ShapeDtypeStruct((N, AF), jnp.float32),
            jax.ShapeDtypeStruct((1, AF), jnp.float32),
            jax.ShapeDtypeStruct((1, AF), jnp.float32),
        ],
    )(g, su, sq, g1, bt1)


def _bn2_update(x_ref, s_ref, ssu_ref, ssq_ref, g2_ref, bt2_ref):
    m2 = ssu_ref[...] / N
    v2 = ssq_ref[...] / N - m2 * m2
    a2 = g2_ref[...] * lax.rsqrt(v2 + 1e-5)
    c2 = bt2_ref[...] - m2 * a2
    return _softplus(x_ref[...] + s_ref[...] * a2 + c2)


def _update_body(x_ref, s_ref, ssu_ref, ssq_ref, g2_ref, bt2_ref, wcat_ref,
                 xo_ref, ps_ref, pn_ref):
    xn = _bn2_update(x_ref, s_ref, ssu_ref, ssq_ref, g2_ref, bt2_ref)
    xo_ref[...] = xn
    p = jnp.dot(xn, wcat_ref[...], preferred_element_type=jnp.float32)
    ps_ref[...] = p[:, :F2]
    pn_ref[...] = p[:, F2:]


def _update(x, s, ssu, ssq, g2, bt2, Wcat):
    return pl.pallas_call(
        _update_body,
        grid=(N // _RT,),
        in_specs=[
            pl.BlockSpec((_RT, AF), lambda i: (i, 0)),
            pl.BlockSpec((_RT, AF), lambda i: (i, 0)),
            pl.BlockSpec((1, AF), lambda i: (0, 0)),
            pl.BlockSpec((1, AF), lambda i: (0, 0)),
            pl.BlockSpec((1, AF), lambda i: (0, 0)),
            pl.BlockSpec((1, AF), lambda i: (0, 0)),
            pl.BlockSpec((AF, 2 * F2), lambda i: (0, 0)),
        ],
        out_specs=[
            pl.BlockSpec((_RT, AF), lambda i: (i, 0)),
            pl.BlockSpec((_RT, F2), lambda i: (i, 0)),
            pl.BlockSpec((_RT, F2), lambda i: (i, 0)),
        ],
        out_shape=[
            jax.ShapeDtypeStruct((N, AF), jnp.float32),
            jax.ShapeDtypeStruct((N, F2), jnp.float32),
            jax.ShapeDtypeStruct((N, F2), jnp.float32),
        ],
    )(x, s, ssu, ssq, g2, bt2, Wcat)


def _update_last_body(x_ref, s_ref, ssu_ref, ssq_ref, g2_ref, bt2_ref, xo_ref):
    xo_ref[...] = _bn2_update(x_ref, s_ref, ssu_ref, ssq_ref, g2_ref, bt2_ref)


def _update_last(x, s, ssu, ssq, g2, bt2):
    return pl.pallas_call(
        _update_last_body,
        grid=(N // _RT,),
        in_specs=[
            pl.BlockSpec((_RT, AF), lambda i: (i, 0)),
            pl.BlockSpec((_RT, AF), lambda i: (i, 0)),
            pl.BlockSpec((1, AF), lambda i: (0, 0)),
            pl.BlockSpec((1, AF), lambda i: (0, 0)),
            pl.BlockSpec((1, AF), lambda i: (0, 0)),
            pl.BlockSpec((1, AF), lambda i: (0, 0)),
        ],
        out_specs=pl.BlockSpec((_RT, AF), lambda i: (i, 0)),
        out_shape=jax.ShapeDtypeStruct((N, AF), jnp.float32),
    )(x, s, ssu, ssq, g2, bt2)


def _final_body(x_ref, w2c_ref, b2_ref, waf_ref, baf_ref,
                ep_ref, af_ref, z_ref, n_ref):
    x = x_ref[...]
    nrm = jnp.sqrt(jnp.sum(x * x, axis=1, keepdims=True))
    nd = x / jnp.maximum(nrm, 1e-12)
    n_ref[...] = nd[None]
    z_ref[...] = jnp.mean(nd, axis=0, keepdims=True)[None]
    af_ref[...] = jnp.dot(nd, waf_ref[...], preferred_element_type=jnp.float32) + baf_ref[...]
    tmp = jnp.dot(nd, w2c_ref[...], preferred_element_type=jnp.float32)
    tmp_r = jnp.concatenate([tmp[:, j * AF:(j + 1) * AF] for j in range(6)], axis=0)
    es_all = lax.dot_general(tmp_r, nd, (((1,), (1,)), ((), ())),
                             preferred_element_type=jnp.float32)
    es = [es_all[j * A:(j + 1) * A, :] + b2_ref[0, j] for j in range(6)]
    mx = es[0]
    for j in range(1, 6):
        mx = jnp.maximum(mx, es[j])
    se = jnp.exp(es[0] - mx)
    for j in range(1, 6):
        se += jnp.exp(es[j] - mx)
    off = mx + jnp.log(se)
    ep_ref[...] = jnp.concatenate([es[j] - off for j in range(6)], axis=0)[None]


def _final(x, W2c, b2, W_af, b_af):
    return pl.pallas_call(
        _final_body,
        grid=(B,),
        in_specs=[
            pl.BlockSpec((A, AF), lambda i: (i, 0)),
            pl.BlockSpec((AF, 6 * AF), lambda i: (0, 0)),
            pl.BlockSpec((1, 6), lambda i: (0, 0)),
            pl.BlockSpec((AF, ORIG), lambda i: (0, 0)),
            pl.BlockSpec((1, ORIG), lambda i: (0, 0)),
        ],
        out_specs=[
            pl.BlockSpec((1, 6 * A, A), lambda i: (i, 0, 0)),
            pl.BlockSpec((A, ORIG), lambda i: (i, 0)),
            pl.BlockSpec((1, 1, AF), lambda i: (i, 0, 0)),
            pl.BlockSpec((1, A, AF), lambda i: (i, 0, 0)),
        ],
        out_shape=[
            jax.ShapeDtypeStruct((B, 6 * A, A), jnp.float32),
            jax.ShapeDtypeStruct((N, ORIG), jnp.float32),
            jax.ShapeDtypeStruct((B, 1, AF), jnp.float32),
            jax.ShapeDtypeStruct((B, A, AF), jnp.float32),
        ],
    )(x, W2c, b2, W_af, b_af)


def kernel(atom_fea, nbr_fea, nbr_fea_idx, crystal_atom_idx, cuda_flag, W_emb,
           Wf0, bf0, g1_0, bt1_0, g2_0, bt2_0,
           Wf1, bf1, g1_1, bt1_1, g2_1, bt2_1,
           Wf2, bf2, g1_2, bt1_2, g2_2, bt2_2,
           W_bil, b_bil, W_fc1, b_fc1, W_af, b_af):
    Wf = [Wf0, Wf1, Wf2]
    g1 = [g1_0[None], g1_1[None], g1_2[None]]
    bt1 = [bt1_0[None], bt1_1[None], bt1_2[None]]
    g2 = [g2_0[None], g2_1[None], g2_2[None]]
    bt2 = [bt2_0[None], bt2_1[None], bt2_2[None]]
    Wcat = [jnp.concatenate([w[:AF], w[AF:2 * AF]], axis=1) for w in Wf]
    Wfe = [w[2 * AF:].astype(jnp.bfloat16) for w in Wf]
    idx = nbr_fea_idx.reshape(-1).astype(jnp.int32)
    nbr_flat = nbr_fea.reshape(K, NBR).astype(jnp.bfloat16)

    x, ps, pn = _embed(atom_fea, W_emb, Wcat[0])
    for l in range(NC):
        an = _sc_gather(pn, idx)
        g, su, sq = _stats(an, nbr_flat, ps, Wfe[l])
        s, ssu, ssq = _act(g, su, sq, g1[l], bt1[l])
        if l + 1 < NC:
            x, ps, pn = _update(x, s, ssu, ssq, g2[l], bt2[l], Wcat[l + 1])
        else:
            x = _update_last(x, s, ssu, ssq, g2[l], bt2[l])

    W2 = jnp.einsum('kde,kj->jde', W_bil, W_fc1)
    W2c = jnp.concatenate([W2[j] for j in range(6)], axis=1)
    b2 = (b_bil @ W_fc1 + b_fc1)[None]
    epk, af, z, normed = _final(x, W2c, b2, W_af, b_af[None])
    ep = jnp.transpose(epk.reshape(B, 6, A, A), (0, 2, 3, 1)).reshape(-1, 6)
    return ep, af, z.reshape(B, AF), normed, x


# speedup vs baseline: 4.3459x; 1.0075x over previous
import functools

import jax
import jax.numpy as jnp
from jax import lax
from jax.experimental import pallas as pl
from jax.experimental.pallas import tpu as pltpu
from jax.experimental.pallas import tpu_sc as plsc

B, A, M = 64, 128, 12
ORIG, NBR, AF, NC = 92, 41, 64, 3
N = B * A
K = N * M
F2 = 2 * AF

_SC_CORES, _SC_SUBCORES = 2, 16
_NW = _SC_CORES * _SC_SUBCORES
_ROWS_PER_W = K // _NW
_CH = 128
_NCH = _ROWS_PER_W // _CH


def _sc_gather(table, idx):
    mesh = plsc.VectorSubcoreMesh(core_axis_name="c", subcore_axis_name="s")

    @functools.partial(
        pl.kernel, mesh=mesh,
        out_type=jax.ShapeDtypeStruct((K, F2), jnp.float32),
        scratch_types=[
            pltpu.VMEM((_ROWS_PER_W,), jnp.int32),
            pltpu.VMEM((_CH, F2), jnp.float32),
            pltpu.VMEM((_CH, F2), jnp.float32),
            pltpu.SemaphoreType.DMA,
            pltpu.SemaphoreType.DMA,
        ],
    )
    def gk(table_hbm, idx_hbm, out_hbm, idx_all, rows0, rows1, sem0, sem1):
        wid = lax.axis_index("s") * _SC_CORES + lax.axis_index("c")
        base_w = pl.multiple_of(wid * _ROWS_PER_W, _ROWS_PER_W)
        rows = (rows0, rows1)
        sems = (sem0, sem1)
        pltpu.sync_copy(idx_hbm.at[pl.ds(base_w, _ROWS_PER_W)], idx_all)
        for b in range(2):
            pltpu.async_copy(
                table_hbm.at[idx_all.at[pl.ds(b * _CH, _CH)]], rows[b], sems[b])

        def body(it, carry):
            for b in range(2):
                j = 2 * it + b
                pltpu.make_async_copy(
                    table_hbm.at[idx_all.at[pl.ds(0, _CH)]], rows[b], sems[b]).wait()
                pltpu.sync_copy(
                    rows[b], out_hbm.at[pl.ds(base_w + j * _CH, _CH)])
                j2 = j + 2

                @pl.when(j2 < _NCH)
                def _():
                    pltpu.async_copy(
                        table_hbm.at[idx_all.at[pl.ds(j2 * _CH, _CH)]],
                        rows[b], sems[b])
            return carry

        lax.fori_loop(0, _NCH // 2, body, 0)

    return gk(table, idx)


def _softplus(z):
    return jnp.log(1.0 + jnp.exp(-jnp.abs(z))) + jnp.maximum(z, 0.0)


def _sigmoid(z):
    return 1.0 / (1.0 + jnp.exp(-z))


_RT = 1024
_NT = 1024
_GT = _NT * M


def _embed_body(af_ref, wemb_ref, wcat_ref, x_ref, ps_ref, pn_ref):
    x = jnp.dot(af_ref[...], wemb_ref[...], preferred_element_type=jnp.float32)
    x_ref[...] = x
    p = jnp.dot(x, wcat_ref[...], preferred_element_type=jnp.float32)
    ps_ref[...] = p[:, :F2]
    pn_ref[...] = p[:, F2:]


def _embed(atom_fea, W_emb, Wcat):
    return pl.pallas_call(
        _embed_body,
        grid=(N // _RT,),
        in_specs=[
            pl.BlockSpec((_RT, ORIG), lambda i: (i, 0)),
            pl.BlockSpec((ORIG, AF), lambda i: (0, 0)),
            pl.BlockSpec((AF, 2 * F2), lambda i: (0, 0)),
        ],
        out_specs=[
            pl.BlockSpec((_RT, AF), lambda i: (i, 0)),
            pl.BlockSpec((_RT, F2), lambda i: (i, 0)),
            pl.BlockSpec((_RT, F2), lambda i: (i, 0)),
        ],
        out_shape=[
            jax.ShapeDtypeStruct((N, AF), jnp.float32),
            jax.ShapeDtypeStruct((N, F2), jnp.float32),
            jax.ShapeDtypeStruct((N, F2), jnp.float32),
        ],
    )(atom_fea, W_emb, Wcat)


def _stats_body(an_ref, nbr_ref, ps_ref, wfe_ref, g_ref, su_ref, sq_ref):
    pe = jnp.dot(nbr_ref[...], wfe_ref[...], preferred_element_type=jnp.float32)
    ps = ps_ref[...]
    psr = jnp.broadcast_to(ps[:, None, :], (_NT, M, F2)).reshape(_GT, F2)
    g = pe + an_ref[...] + psr
    g_ref[...] = g.astype(jnp.bfloat16)

    @pl.when(pl.program_id(0) == 0)
    def _():
        su_ref[...] = jnp.zeros_like(su_ref)
        sq_ref[...] = jnp.zeros_like(sq_ref)

    su_ref[...] += jnp.sum(g, axis=0, keepdims=True)
    sq_ref[...] += jnp.sum(g * g, axis=0, keepdims=True)


def _stats(an, nbr_flat, ps, Wfe):
    return pl.pallas_call(
        _stats_body,
        grid=(N // _NT,),
        in_specs=[
            pl.BlockSpec((_GT, F2), lambda i: (i, 0)),
            pl.BlockSpec((_GT, NBR), lambda i: (i, 0)),
            pl.BlockSpec((_NT, F2), lambda i: (i, 0)),
            pl.BlockSpec((NBR, F2), lambda i: (0, 0)),
        ],
        out_specs=[
            pl.BlockSpec((_GT, F2), lambda i: (i, 0)),
            pl.BlockSpec((1, F2), lambda i: (0, 0)),
            pl.BlockSpec((1, F2), lambda i: (0, 0)),
        ],
        out_shape=[
            jax.ShapeDtypeStruct((K, F2), jnp.bfloat16),
            jax.ShapeDtypeStruct((1, F2), jnp.float32),
            jax.ShapeDtypeStruct((1, F2), jnp.float32),
        ],
    )(an, nbr_flat, ps, Wfe)


def _act_body(g_ref, su_ref, sq_ref, g1_ref, bt1_ref, s_ref, ssu_ref, ssq_ref):
    mean = su_ref[...] / K
    var = sq_ref[...] / K - mean * mean
    a = g1_ref[...] * lax.rsqrt(var + 1e-5)
    c = bt1_ref[...] - mean * a
    g = g_ref[...].astype(jnp.float32)
    gh = g * a + c
    fl = _sigmoid(gh[:, :AF])
    co = _softplus(gh[:, AF:])
    s = jnp.sum((fl * co).reshape(_NT, M, AF), axis=1)
    s_ref[...] = s

    @pl.when(pl.program_id(0) == 0)
    def _():
        ssu_ref[...] = jnp.zeros_like(ssu_ref)
        ssq_ref[...] = jnp.zeros_like(ssq_ref)

    ssu_ref[...] += jnp.sum(s, axis=0, keepdims=True)
    ssq_ref[...] += jnp.sum(s * s, axis=0, keepdims=True)


def _act(g, su, sq, g1, bt1):
    return pl.pallas_call(
        _act_body,
        grid=(N // _NT,),
        in_specs=[
            pl.BlockSpec((_GT, F2), lambda i: (i, 0)),
            pl.BlockSpec((1, F2), lambda i: (0, 0)),
            pl.BlockSpec((1, F2), lambda i: (0, 0)),
            pl.BlockSpec((1, F2), lambda i: (0, 0)),
            pl.BlockSpec((1, F2), lambda i: (0, 0)),
        ],
        out_specs=[
            pl.BlockSpec((_NT, AF), lambda i: (i, 0)),
            pl.BlockSpec((1, AF), lambda i: (0, 0)),
            pl.BlockSpec((1, AF), lambda i: (0, 0)),
        ],
        out_shape=[
            jax.ShapeDtypeStruct((N, AF), jnp.float32),
            jax.ShapeDtypeStruct((1, AF), jnp.float32),
            jax.ShapeDtypeStruct((1, AF), jnp.float32),
        ],
    )(g, su, sq, g1, bt1)


def _bn2_update(x_ref, s_ref, ssu_ref, ssq_ref, g2_ref, bt2_ref):
    m2 = ssu_ref[...] / N
    v2 = ssq_ref[...] / N - m2 * m2
    a2 = g2_ref[...] * lax.rsqrt(v2 + 1e-5)
    c2 = bt2_ref[...] - m2 * a2
    return _softplus(x_ref[...] + s_ref[...] * a2 + c2)


def _update_body(x_ref, s_ref, ssu_ref, ssq_ref, g2_ref, bt2_ref, wcat_ref,
                 xo_ref, ps_ref, pn_ref):
    xn = _bn2_update(x_ref, s_ref, ssu_ref, ssq_ref, g2_ref, bt2_ref)
    xo_ref[...] = xn
    p = jnp.dot(xn, wcat_ref[...], preferred_element_type=jnp.float32)
    ps_ref[...] = p[:, :F2]
    pn_ref[...] = p[:, F2:]


def _update(x, s, ssu, ssq, g2, bt2, Wcat):
    return pl.pallas_call(
        _update_body,
        grid=(N // _RT,),
        in_specs=[
            pl.BlockSpec((_RT, AF), lambda i: (i, 0)),
            pl.BlockSpec((_RT, AF), lambda i: (i, 0)),
            pl.BlockSpec((1, AF), lambda i: (0, 0)),
            pl.BlockSpec((1, AF), lambda i: (0, 0)),
            pl.BlockSpec((1, AF), lambda i: (0, 0)),
            pl.BlockSpec((1, AF), lambda i: (0, 0)),
            pl.BlockSpec((AF, 2 * F2), lambda i: (0, 0)),
        ],
        out_specs=[
            pl.BlockSpec((_RT, AF), lambda i: (i, 0)),
            pl.BlockSpec((_RT, F2), lambda i: (i, 0)),
            pl.BlockSpec((_RT, F2), lambda i: (i, 0)),
        ],
        out_shape=[
            jax.ShapeDtypeStruct((N, AF), jnp.float32),
            jax.ShapeDtypeStruct((N, F2), jnp.float32),
            jax.ShapeDtypeStruct((N, F2), jnp.float32),
        ],
    )(x, s, ssu, ssq, g2, bt2, Wcat)


def _update_last_body(x_ref, s_ref, ssu_ref, ssq_ref, g2_ref, bt2_ref, xo_ref):
    xo_ref[...] = _bn2_update(x_ref, s_ref, ssu_ref, ssq_ref, g2_ref, bt2_ref)


def _update_last(x, s, ssu, ssq, g2, bt2):
    return pl.pallas_call(
        _update_last_body,
        grid=(N // _RT,),
        in_specs=[
            pl.BlockSpec((_RT, AF), lambda i: (i, 0)),
            pl.BlockSpec((_RT, AF), lambda i: (i, 0)),
            pl.BlockSpec((1, AF), lambda i: (0, 0)),
            pl.BlockSpec((1, AF), lambda i: (0, 0)),
            pl.BlockSpec((1, AF), lambda i: (0, 0)),
            pl.BlockSpec((1, AF), lambda i: (0, 0)),
        ],
        out_specs=pl.BlockSpec((_RT, AF), lambda i: (i, 0)),
        out_shape=jax.ShapeDtypeStruct((N, AF), jnp.float32),
    )(x, s, ssu, ssq, g2, bt2)


def _final_body(x_ref, w2c_ref, b2_ref, waf_ref, baf_ref,
                ep_ref, af_ref, z_ref, n_ref):
    x = x_ref[...]
    nrm = jnp.sqrt(jnp.sum(x * x, axis=1, keepdims=True))
    nd = x / jnp.maximum(nrm, 1e-12)
    n_ref[...] = nd[None]
    z_ref[...] = jnp.mean(nd, axis=0, keepdims=True)[None]
    af_ref[...] = jnp.dot(nd, waf_ref[...], preferred_element_type=jnp.float32) + baf_ref[...]
    tmp = jnp.dot(nd, w2c_ref[...], preferred_element_type=jnp.float32)
    tmp_r = jnp.concatenate([tmp[:, j * AF:(j + 1) * AF] for j in range(6)], axis=0)
    es_all = lax.dot_general(tmp_r, nd, (((1,), (1,)), ((), ())),
                             preferred_element_type=jnp.float32)
    es = [es_all[j * A:(j + 1) * A, :] + b2_ref[0, j] for j in range(6)]
    mx = es[0]
    for j in range(1, 6):
        mx = jnp.maximum(mx, es[j])
    se = jnp.exp(es[0] - mx)
    for j in range(1, 6):
        se += jnp.exp(es[j] - mx)
    off = mx + jnp.log(se)
    ep_ref[...] = jnp.concatenate([es[j] - off for j in range(6)], axis=0)[None]


def _final(x, W2c, b2, W_af, b_af):
    return pl.pallas_call(
        _final_body,
        grid=(B,),
        in_specs=[
            pl.BlockSpec((A, AF), lambda i: (i, 0)),
            pl.BlockSpec((AF, 6 * AF), lambda i: (0, 0)),
            pl.BlockSpec((1, 6), lambda i: (0, 0)),
            pl.BlockSpec((AF, ORIG), lambda i: (0, 0)),
            pl.BlockSpec((1, ORIG), lambda i: (0, 0)),
        ],
        out_specs=[
            pl.BlockSpec((1, 6 * A, A), lambda i: (i, 0, 0)),
            pl.BlockSpec((A, ORIG), lambda i: (i, 0)),
            pl.BlockSpec((1, 1, AF), lambda i: (i, 0, 0)),
            pl.BlockSpec((1, A, AF), lambda i: (i, 0, 0)),
        ],
        out_shape=[
            jax.ShapeDtypeStruct((B, 6 * A, A), jnp.float32),
            jax.ShapeDtypeStruct((N, ORIG), jnp.float32),
            jax.ShapeDtypeStruct((B, 1, AF), jnp.float32),
            jax.ShapeDtypeStruct((B, A, AF), jnp.float32),
        ],
    )(x, W2c, b2, W_af, b_af)


def kernel(atom_fea, nbr_fea, nbr_fea_idx, crystal_atom_idx, cuda_flag, W_emb,
           Wf0, bf0, g1_0, bt1_0, g2_0, bt2_0,
           Wf1, bf1, g1_1, bt1_1, g2_1, bt2_1,
           Wf2, bf2, g1_2, bt1_2, g2_2, bt2_2,
           W_bil, b_bil, W_fc1, b_fc1, W_af, b_af):
    Wf = [Wf0, Wf1, Wf2]
    g1 = [g1_0[None], g1_1[None], g1_2[None]]
    bt1 = [bt1_0[None], bt1_1[None], bt1_2[None]]
    g2 = [g2_0[None], g2_1[None], g2_2[None]]
    bt2 = [bt2_0[None], bt2_1[None], bt2_2[None]]
    Wcat = [jnp.concatenate([w[:AF], w[AF:2 * AF]], axis=1) for w in Wf]
    Wfe = [w[2 * AF:].astype(jnp.bfloat16) for w in Wf]
    idx = nbr_fea_idx.reshape(-1).astype(jnp.int32)
    nbr_flat = nbr_fea.reshape(K, NBR).astype(jnp.bfloat16)

    x, ps, pn = _embed(atom_fea, W_emb, Wcat[0])
    for l in range(NC):
        an = _sc_gather(pn, idx)
        g, su, sq = _stats(an, nbr_flat, ps, Wfe[l])
        s, ssu, ssq = _act(g, su, sq, g1[l], bt1[l])
        if l + 1 < NC:
            x, ps, pn = _update(x, s, ssu, ssq, g2[l], bt2[l], Wcat[l + 1])
        else:
            x = _update_last(x, s, ssu, ssq, g2[l], bt2[l])

    W2 = jnp.einsum('kde,kj->jde', W_bil, W_fc1)
    W2c = jnp.concatenate([W2[j] for j in range(6)], axis=1)
    b2 = (b_bil @ W_fc1 + b_fc1)[None]
    epk, af, z, normed = _final(x, W2c, b2, W_af, b_af[None])
    ep = jnp.transpose(epk.reshape(B, 6, A, A), (0, 2, 3, 1)).reshape(-1, 6)
    return ep, af, z.reshape(B, AF), normed, x


# speedup vs baseline: 5.0204x; 1.1552x over previous
import functools

import jax
import jax.numpy as jnp
from jax import lax
from jax.experimental import pallas as pl
from jax.experimental.pallas import tpu as pltpu
from jax.experimental.pallas import tpu_sc as plsc

B, A, M = 64, 128, 12
ORIG, NBR, AF, NC = 92, 41, 64, 3
N = B * A
K = N * M
F2 = 2 * AF

_SC_CORES, _SC_SUBCORES = 2, 16
_NW = _SC_CORES * _SC_SUBCORES
_ROWS_PER_W = K // _NW
_CH = 128
_NCH = _ROWS_PER_W // _CH


def _sc_gather(table, idx):
    mesh = plsc.VectorSubcoreMesh(core_axis_name="c", subcore_axis_name="s")

    @functools.partial(
        pl.kernel, mesh=mesh,
        out_type=jax.ShapeDtypeStruct((K, F2), jnp.float32),
        scratch_types=[
            pltpu.VMEM((_ROWS_PER_W,), jnp.int32),
            pltpu.VMEM((_CH, F2), jnp.float32),
            pltpu.VMEM((_CH, F2), jnp.float32),
            pltpu.SemaphoreType.DMA,
            pltpu.SemaphoreType.DMA,
        ],
    )
    def gk(table_hbm, idx_hbm, out_hbm, idx_all, rows0, rows1, sem0, sem1):
        wid = lax.axis_index("s") * _SC_CORES + lax.axis_index("c")
        base_w = pl.multiple_of(wid * _ROWS_PER_W, _ROWS_PER_W)
        rows = (rows0, rows1)
        sems = (sem0, sem1)
        pltpu.sync_copy(idx_hbm.at[pl.ds(base_w, _ROWS_PER_W)], idx_all)
        for b in range(2):
            pltpu.async_copy(
                table_hbm.at[idx_all.at[pl.ds(b * _CH, _CH)]], rows[b], sems[b])

        def body(it, carry):
            for b in range(2):
                j = 2 * it + b
                pltpu.make_async_copy(
                    table_hbm.at[idx_all.at[pl.ds(0, _CH)]], rows[b], sems[b]).wait()
                pltpu.sync_copy(
                    rows[b], out_hbm.at[pl.ds(base_w + j * _CH, _CH)])
                j2 = j + 2

                @pl.when(j2 < _NCH)
                def _():
                    pltpu.async_copy(
                        table_hbm.at[idx_all.at[pl.ds(j2 * _CH, _CH)]],
                        rows[b], sems[b])
            return carry

        lax.fori_loop(0, _NCH // 2, body, 0)

    return gk(table, idx)


def _softplus(z):
    return jnp.log(1.0 + jnp.exp(-jnp.abs(z))) + jnp.maximum(z, 0.0)


def _sigmoid(z):
    return 0.5 * jnp.tanh(0.5 * z) + 0.5


_RT = 1024
_NT = 1024
_GT = _NT * M


def _embed_body(af_ref, wemb_ref, wcat_ref, x_ref, ps_ref, pn_ref):
    x = jnp.dot(af_ref[...], wemb_ref[...], preferred_element_type=jnp.float32)
    x_ref[...] = x
    p = jnp.dot(x, wcat_ref[...], preferred_element_type=jnp.float32)
    ps_ref[...] = p[:, :F2]
    pn_ref[...] = p[:, F2:]


def _embed(atom_fea, W_emb, Wcat):
    return pl.pallas_call(
        _embed_body,
        grid=(N // _RT,),
        in_specs=[
            pl.BlockSpec((_RT, ORIG), lambda i: (i, 0)),
            pl.BlockSpec((ORIG, AF), lambda i: (0, 0)),
            pl.BlockSpec((AF, 2 * F2), lambda i: (0, 0)),
        ],
        out_specs=[
            pl.BlockSpec((_RT, AF), lambda i: (i, 0)),
            pl.BlockSpec((_RT, F2), lambda i: (i, 0)),
            pl.BlockSpec((_RT, F2), lambda i: (i, 0)),
        ],
        out_shape=[
            jax.ShapeDtypeStruct((N, AF), jnp.float32),
            jax.ShapeDtypeStruct((N, F2), jnp.float32),
            jax.ShapeDtypeStruct((N, F2), jnp.float32),
        ],
    )(atom_fea, W_emb, Wcat)


def _stats_body(an_ref, nbr_ref, ps_ref, wfe_ref, g_ref, su_ref, sq_ref):
    nbr = nbr_ref[...].reshape(M * _NT, NBR)
    pe = jnp.dot(nbr, wfe_ref[...], preferred_element_type=jnp.float32)
    psr = jnp.broadcast_to(ps_ref[...][None], (M, _NT, F2)).reshape(M * _NT, F2)
    g = pe + an_ref[...].reshape(M * _NT, F2) + psr
    g_ref[...] = g.astype(jnp.bfloat16).reshape(M, _NT, F2)

    @pl.when(pl.program_id(0) == 0)
    def _():
        su_ref[...] = jnp.zeros_like(su_ref)
        sq_ref[...] = jnp.zeros_like(sq_ref)

    su_ref[...] += jnp.sum(g, axis=0, keepdims=True)
    sq_ref[...] += jnp.sum(g * g, axis=0, keepdims=True)


def _stats(an3, nbr3, ps, Wfe):
    return pl.pallas_call(
        _stats_body,
        grid=(N // _NT,),
        in_specs=[
            pl.BlockSpec((M, _NT, F2), lambda i: (0, i, 0)),
            pl.BlockSpec((M, _NT, NBR), lambda i: (0, i, 0)),
            pl.BlockSpec((_NT, F2), lambda i: (i, 0)),
            pl.BlockSpec((NBR, F2), lambda i: (0, 0)),
        ],
        out_specs=[
            pl.BlockSpec((M, _NT, F2), lambda i: (0, i, 0)),
            pl.BlockSpec((1, F2), lambda i: (0, 0)),
            pl.BlockSpec((1, F2), lambda i: (0, 0)),
        ],
        out_shape=[
            jax.ShapeDtypeStruct((M, N, F2), jnp.bfloat16),
            jax.ShapeDtypeStruct((1, F2), jnp.float32),
            jax.ShapeDtypeStruct((1, F2), jnp.float32),
        ],
    )(an3, nbr3, ps, Wfe)


def _act_body(g_ref, su_ref, sq_ref, g1_ref, bt1_ref, s_ref, ssu_ref, ssq_ref):
    mean = su_ref[...] / K
    var = sq_ref[...] / K - mean * mean
    a = g1_ref[...] * lax.rsqrt(var + 1e-5)
    c = bt1_ref[...] - mean * a
    g = g_ref[...].astype(jnp.float32).reshape(M * _NT, F2)
    gh = g * a + c
    fl = _sigmoid(gh[:, :AF])
    co = _softplus(gh[:, AF:])
    prod = (fl * co).reshape(M, _NT, AF)
    s = prod[0]
    for m in range(1, M):
        s = s + prod[m]
    s_ref[...] = s

    @pl.when(pl.program_id(0) == 0)
    def _():
        ssu_ref[...] = jnp.zeros_like(ssu_ref)
        ssq_ref[...] = jnp.zeros_like(ssq_ref)

    ssu_ref[...] += jnp.sum(s, axis=0, keepdims=True)
    ssq_ref[...] += jnp.sum(s * s, axis=0, keepdims=True)


def _act(g, su, sq, g1, bt1):
    return pl.pallas_call(
        _act_body,
        grid=(N // _NT,),
        in_specs=[
            pl.BlockSpec((M, _NT, F2), lambda i: (0, i, 0)),
            pl.BlockSpec((1, F2), lambda i: (0, 0)),
            pl.BlockSpec((1, F2), lambda i: (0, 0)),
            pl.BlockSpec((1, F2), lambda i: (0, 0)),
            pl.BlockSpec((1, F2), lambda i: (0, 0)),
        ],
        out_specs=[
            pl.BlockSpec((_NT, AF), lambda i: (i, 0)),
            pl.BlockSpec((1, AF), lambda i: (0, 0)),
            pl.BlockSpec((1, AF), lambda i: (0, 0)),
        ],
        out_shape=[
            jax.ShapeDtypeStruct((N, AF), jnp.float32),
            jax.ShapeDtypeStruct((1, AF), jnp.float32),
            jax.ShapeDtypeStruct((1, AF), jnp.float32),
        ],
    )(g, su, sq, g1, bt1)


def _bn2_update(x_ref, s_ref, ssu_ref, ssq_ref, g2_ref, bt2_ref):
    m2 = ssu_ref[...] / N
    v2 = ssq_ref[...] / N - m2 * m2
    a2 = g2_ref[...] * lax.rsqrt(v2 + 1e-5)
    c2 = bt2_ref[...] - m2 * a2
    return _softplus(x_ref[...] + s_ref[...] * a2 + c2)


def _update_body(x_ref, s_ref, ssu_ref, ssq_ref, g2_ref, bt2_ref, wcat_ref,
                 xo_ref, ps_ref, pn_ref):
    xn = _bn2_update(x_ref, s_ref, ssu_ref, ssq_ref, g2_ref, bt2_ref)
    xo_ref[...] = xn
    p = jnp.dot(xn, wcat_ref[...], preferred_element_type=jnp.float32)
    ps_ref[...] = p[:, :F2]
    pn_ref[...] = p[:, F2:]


def _update(x, s, ssu, ssq, g2, bt2, Wcat):
    return pl.pallas_call(
        _update_body,
        grid=(N // _RT,),
        in_specs=[
            pl.BlockSpec((_RT, AF), lambda i: (i, 0)),
            pl.BlockSpec((_RT, AF), lambda i: (i, 0)),
            pl.BlockSpec((1, AF), lambda i: (0, 0)),
            pl.BlockSpec((1, AF), lambda i: (0, 0)),
            pl.BlockSpec((1, AF), lambda i: (0, 0)),
            pl.BlockSpec((1, AF), lambda i: (0, 0)),
            pl.BlockSpec((AF, 2 * F2), lambda i: (0, 0)),
        ],
        out_specs=[
            pl.BlockSpec((_RT, AF), lambda i: (i, 0)),
            pl.BlockSpec((_RT, F2), lambda i: (i, 0)),
            pl.BlockSpec((_RT, F2), lambda i: (i, 0)),
        ],
        out_shape=[
            jax.ShapeDtypeStruct((N, AF), jnp.float32),
            jax.ShapeDtypeStruct((N, F2), jnp.float32),
            jax.ShapeDtypeStruct((N, F2), jnp.float32),
        ],
    )(x, s, ssu, ssq, g2, bt2, Wcat)


def _update_last_body(x_ref, s_ref, ssu_ref, ssq_ref, g2_ref, bt2_ref, xo_ref):
    xo_ref[...] = _bn2_update(x_ref, s_ref, ssu_ref, ssq_ref, g2_ref, bt2_ref)


def _update_last(x, s, ssu, ssq, g2, bt2):
    return pl.pallas_call(
        _update_last_body,
        grid=(N // _RT,),
        in_specs=[
            pl.BlockSpec((_RT, AF), lambda i: (i, 0)),
            pl.BlockSpec((_RT, AF), lambda i: (i, 0)),
            pl.BlockSpec((1, AF), lambda i: (0, 0)),
            pl.BlockSpec((1, AF), lambda i: (0, 0)),
            pl.BlockSpec((1, AF), lambda i: (0, 0)),
            pl.BlockSpec((1, AF), lambda i: (0, 0)),
        ],
        out_specs=pl.BlockSpec((_RT, AF), lambda i: (i, 0)),
        out_shape=jax.ShapeDtypeStruct((N, AF), jnp.float32),
    )(x, s, ssu, ssq, g2, bt2)


def _final_body(x_ref, w2c_ref, b2_ref, waf_ref, baf_ref,
                ep_ref, af_ref, z_ref, n_ref):
    x = x_ref[...]
    nrm = jnp.sqrt(jnp.sum(x * x, axis=1, keepdims=True))
    nd = x / jnp.maximum(nrm, 1e-12)
    n_ref[...] = nd[None]
    z_ref[...] = jnp.mean(nd, axis=0, keepdims=True)[None]
    af_ref[...] = jnp.dot(nd, waf_ref[...], preferred_element_type=jnp.float32) + baf_ref[...]
    tmp = jnp.dot(nd, w2c_ref[...], preferred_element_type=jnp.float32)
    tmp_r = jnp.concatenate([tmp[:, j * AF:(j + 1) * AF] for j in range(6)], axis=0)
    es_all = lax.dot_general(tmp_r, nd, (((1,), (1,)), ((), ())),
                             preferred_element_type=jnp.float32)
    es = [es_all[j * A:(j + 1) * A, :] + b2_ref[0, j] for j in range(6)]
    mx = es[0]
    for j in range(1, 6):
        mx = jnp.maximum(mx, es[j])
    se = jnp.exp(es[0] - mx)
    for j in range(1, 6):
        se += jnp.exp(es[j] - mx)
    off = mx + jnp.log(se)
    ep_ref[...] = jnp.concatenate([es[j] - off for j in range(6)], axis=0)[None]


def _final(x, W2c, b2, W_af, b_af):
    return pl.pallas_call(
        _final_body,
        grid=(B,),
        in_specs=[
            pl.BlockSpec((A, AF), lambda i: (i, 0)),
            pl.BlockSpec((AF, 6 * AF), lambda i: (0, 0)),
            pl.BlockSpec((1, 6), lambda i: (0, 0)),
            pl.BlockSpec((AF, ORIG), lambda i: (0, 0)),
            pl.BlockSpec((1, ORIG), lambda i: (0, 0)),
        ],
        out_specs=[
            pl.BlockSpec((1, 6 * A, A), lambda i: (i, 0, 0)),
            pl.BlockSpec((A, ORIG), lambda i: (i, 0)),
            pl.BlockSpec((1, 1, AF), lambda i: (i, 0, 0)),
            pl.BlockSpec((1, A, AF), lambda i: (i, 0, 0)),
        ],
        out_shape=[
            jax.ShapeDtypeStruct((B, 6 * A, A), jnp.float32),
            jax.ShapeDtypeStruct((N, ORIG), jnp.float32),
            jax.ShapeDtypeStruct((B, 1, AF), jnp.float32),
            jax.ShapeDtypeStruct((B, A, AF), jnp.float32),
        ],
    )(x, W2c, b2, W_af, b_af)


def kernel(atom_fea, nbr_fea, nbr_fea_idx, crystal_atom_idx, cuda_flag, W_emb,
           Wf0, bf0, g1_0, bt1_0, g2_0, bt2_0,
           Wf1, bf1, g1_1, bt1_1, g2_1, bt2_1,
           Wf2, bf2, g1_2, bt1_2, g2_2, bt2_2,
           W_bil, b_bil, W_fc1, b_fc1, W_af, b_af):
    Wf = [Wf0, Wf1, Wf2]
    g1 = [g1_0[None], g1_1[None], g1_2[None]]
    bt1 = [bt1_0[None], bt1_1[None], bt1_2[None]]
    g2 = [g2_0[None], g2_1[None], g2_2[None]]
    bt2 = [bt2_0[None], bt2_1[None], bt2_2[None]]
    Wcat = [jnp.concatenate([w[:AF], w[AF:2 * AF]], axis=1) for w in Wf]
    Wfe = [w[2 * AF:].astype(jnp.bfloat16) for w in Wf]
    idx = nbr_fea_idx.T.reshape(-1).astype(jnp.int32)
    nbr3 = nbr_fea.transpose(1, 0, 2).astype(jnp.bfloat16)

    x, ps, pn = _embed(atom_fea, W_emb, Wcat[0])
    for l in range(NC):
        an3 = _sc_gather(pn, idx).reshape(M, N, F2)
        g, su, sq = _stats(an3, nbr3, ps, Wfe[l])
        s, ssu, ssq = _act(g, su, sq, g1[l], bt1[l])
        if l + 1 < NC:
            x, ps, pn = _update(x, s, ssu, ssq, g2[l], bt2[l], Wcat[l + 1])
        else:
            x = _update_last(x, s, ssu, ssq, g2[l], bt2[l])

    W2 = jnp.einsum('kde,kj->jde', W_bil, W_fc1)
    W2c = jnp.concatenate([W2[j] for j in range(6)], axis=1)
    b2 = (b_bil @ W_fc1 + b_fc1)[None]
    epk, af, z, normed = _final(x, W2c, b2, W_af, b_af[None])
    ep = jnp.transpose(epk.reshape(B, 6, A, A), (0, 2, 3, 1)).reshape(-1, 6)
    return ep, af, z.reshape(B, AF), normed, x
